# Initial kernel scaffold; baseline (speedup 1.0000x reference)
import jax
import jax.numpy as jnp
from jax.experimental import pallas as pl


def kernel(x_cat, x_cont, edge_index, edge_attr, pe, batch, params):
    raise NotImplementedError("write your pallas kernel here")


# speedup vs baseline: 9.4827x; 9.4827x over previous
import functools
import jax
import jax.numpy as jnp
from jax.experimental import pallas as pl

N, E, G = 10000, 320000, 128
H, DH, D, L = 8, 16, 128, 4
BE = 2000
SCALE = 0.25


def _group_mat(dtype=jnp.float32):
    d_i = jax.lax.broadcasted_iota(jnp.int32, (D, H), 0)
    h_i = jax.lax.broadcasted_iota(jnp.int32, (D, H), 1)
    return (d_i // DH == h_i).astype(dtype)


def _edge_init_body(ee_ref, w_ref, b_ref, out_ref):
    out_ref[...] = jnp.dot(ee_ref[...], w_ref[...],
                           preferred_element_type=jnp.float32) + b_ref[...]


def edge_init(ee, w, b):
    return pl.pallas_call(
        _edge_init_body,
        grid=(E // BE,),
        in_specs=[
            pl.BlockSpec((BE, 16), lambda i: (i, 0)),
            pl.BlockSpec((16, D), lambda i: (0, 0)),
            pl.BlockSpec((1, D), lambda i: (0, 0)),
        ],
        out_specs=pl.BlockSpec((BE, D), lambda i: (i, 0)),
        out_shape=jax.ShapeDtypeStruct((E, D), jnp.float32),
    )(ee, w, b)


def _epr_body(t2_ref, aff_ref, w_ref, out_ref):
    el = t2_ref[...] * aff_ref[0:1, :] + aff_ref[1:2, :]
    out_ref[...] = jnp.dot(el, w_ref[...], preferred_element_type=jnp.float32)


def epr_kernel(t2, aff, w):
    return pl.pallas_call(
        _epr_body,
        grid=(E // BE,),
        in_specs=[
            pl.BlockSpec((BE, D), lambda i: (i, 0)),
            pl.BlockSpec((2, D), lambda i: (0, 0)),
            pl.BlockSpec((D, D), lambda i: (0, 0)),
        ],
        out_specs=pl.BlockSpec((BE, D), lambda i: (i, 0)),
        out_shape=jax.ShapeDtypeStruct((E, D), jnp.float32),
    )(t2, aff, w)


def _score_fuse_body(qd_ref, ks_ref, vs_ref, epr_ref, t2_ref, aff_ref, woe_ref,
                     exv_ref, ex16_ref, t1_ref, st_ref):
    qk = qd_ref[...] * ks_ref[...] * epr_ref[...] * SCALE
    gm = _group_mat()
    score = jnp.dot(qk, gm, preferred_element_type=jnp.float32)
    ex = jnp.exp(score)
    exb = jnp.dot(ex, gm.T, preferred_element_type=jnp.float32)
    exv_ref[...] = exb * vs_ref[...]
    ex16_ref[...] = jnp.concatenate(
        [ex, jnp.zeros((BE, 8), jnp.float32)], axis=1)
    el = t2_ref[...] * aff_ref[0:1, :] + aff_ref[1:2, :]
    t1 = el + jnp.dot(qk, woe_ref[...], preferred_element_type=jnp.float32)
    t1_ref[...] = t1
    s = jnp.sum(t1, axis=0, keepdims=True)
    ss = jnp.sum(t1 * t1, axis=0, keepdims=True)
    blk = jnp.concatenate(
        [s, ss, jnp.zeros((6, D), jnp.float32)], axis=0)

    @pl.when(pl.program_id(0) == 0)
    def _():
        st_ref[...] = jnp.zeros_like(st_ref)

    st_ref[...] += blk


def score_fuse(qd, ks, vs, epr, t2, aff, woe):
    return pl.pallas_call(
        _score_fuse_body,
        grid=(E // BE,),
        in_specs=[
            pl.BlockSpec((BE, D), lambda i: (i, 0)),
            pl.BlockSpec((BE, D), lambda i: (i, 0)),
            pl.BlockSpec((BE, D), lambda i: (i, 0)),
            pl.BlockSpec((BE, D), lambda i: (i, 0)),
            pl.BlockSpec((BE, D), lambda i: (i, 0)),
            pl.BlockSpec((2, D), lambda i: (0, 0)),
            pl.BlockSpec((D, D), lambda i: (0, 0)),
        ],
        out_specs=[
            pl.BlockSpec((BE, D), lambda i: (i, 0)),
            pl.BlockSpec((BE, 16), lambda i: (i, 0)),
            pl.BlockSpec((BE, D), lambda i: (i, 0)),
            pl.BlockSpec((8, D), lambda i: (0, 0)),
        ],
        out_shape=[
            jax.ShapeDtypeStruct((E, D), jnp.float32),
            jax.ShapeDtypeStruct((E, 16), jnp.float32),
            jax.ShapeDtypeStruct((E, D), jnp.float32),
            jax.ShapeDtypeStruct((8, D), jnp.float32),
        ],
    )(qd, ks, vs, epr, t2, aff, woe)


def _edge_ffn_body(t1_ref, aff_ref, w1_ref, b1_ref, w2_ref, b2_ref,
                   t2_ref, st_ref):
    ep = t1_ref[...] * aff_ref[0:1, :] + aff_ref[1:2, :]
    he = jnp.maximum(
        jnp.dot(ep, w1_ref[...], preferred_element_type=jnp.float32)
        + b1_ref[...], 0.0)
    t2 = ep + jnp.dot(he, w2_ref[...],
                      preferred_element_type=jnp.float32) + b2_ref[...]
    t2_ref[...] = t2
    s = jnp.sum(t2, axis=0, keepdims=True)
    ss = jnp.sum(t2 * t2, axis=0, keepdims=True)
    blk = jnp.concatenate([s, ss, jnp.zeros((6, D), jnp.float32)], axis=0)

    @pl.when(pl.program_id(0) == 0)
    def _():
        st_ref[...] = jnp.zeros_like(st_ref)

    st_ref[...] += blk


def edge_ffn(t1, aff, w1, b1, w2, b2):
    return pl.pallas_call(
        _edge_ffn_body,
        grid=(E // BE,),
        in_specs=[
            pl.BlockSpec((BE, D), lambda i: (i, 0)),
            pl.BlockSpec((2, D), lambda i: (0, 0)),
            pl.BlockSpec((D, 2 * D), lambda i: (0, 0)),
            pl.BlockSpec((1, 2 * D), lambda i: (0, 0)),
            pl.BlockSpec((2 * D, D), lambda i: (0, 0)),
            pl.BlockSpec((1, D), lambda i: (0, 0)),
        ],
        out_specs=[
            pl.BlockSpec((BE, D), lambda i: (i, 0)),
            pl.BlockSpec((8, D), lambda i: (0, 0)),
        ],
        out_shape=[
            jax.ShapeDtypeStruct((E, D), jnp.float32),
            jax.ShapeDtypeStruct((8, D), jnp.float32),
        ],
    )(t1, aff, w1, b1, w2, b2)


def _bn_exact(t, g, b):
    m = jnp.mean(t, axis=0, keepdims=True)
    v = jnp.mean(t * t, axis=0, keepdims=True) - m * m
    return (t - m) * jax.lax.rsqrt(v + 1e-5) * g + b


def _node_prologue_body(xf_ref, pe_ref, wn_ref, bn_ref, wpe_ref,
                        wq_ref, wk_ref, wv_ref,
                        x_ref, q_ref, k_ref, v_ref):
    x = (jnp.dot(xf_ref[...], wn_ref[...], preferred_element_type=jnp.float32)
         + bn_ref[...]
         + jnp.dot(pe_ref[...], wpe_ref[...],
                   preferred_element_type=jnp.float32))
    x_ref[...] = x
    q_ref[...] = jnp.dot(x, wq_ref[...], preferred_element_type=jnp.float32)
    k_ref[...] = jnp.dot(x, wk_ref[...], preferred_element_type=jnp.float32)
    v_ref[...] = jnp.dot(x, wv_ref[...], preferred_element_type=jnp.float32)


def node_prologue(xf, pe, wn, bn, wpe, wq, wk, wv):
    return pl.pallas_call(
        _node_prologue_body,
        out_shape=[jax.ShapeDtypeStruct((N, D), jnp.float32)] * 4,
    )(xf, pe, wn, bn, wpe, wq, wk, wv)


def _node_update_body(x_ref, num_ref, den_ref, wo_ref, g1_ref, b1_ref,
                      w1_ref, bb1_ref, w2_ref, bb2_ref, g2_ref, b2_ref,
                      wq_ref, wk_ref, wv_ref,
                      x_out_ref, q_ref, k_ref, v_ref):
    gm = _group_mat()
    denb = jnp.dot(den_ref[:, 0:H], gm.T,
                   preferred_element_type=jnp.float32)
    agg = num_ref[...] / (denb + 1e-16)
    t = x_ref[...] + jnp.dot(agg, wo_ref[...],
                             preferred_element_type=jnp.float32)
    x1 = _bn_exact(t, g1_ref[...], b1_ref[...])
    h = jnp.maximum(
        jnp.dot(x1, w1_ref[...], preferred_element_type=jnp.float32)
        + bb1_ref[...], 0.0)
    t = x1 + jnp.dot(h, w2_ref[...],
                     preferred_element_type=jnp.float32) + bb2_ref[...]
    x2 = _bn_exact(t, g2_ref[...], b2_ref[...])
    x_out_ref[...] = x2
    q_ref[...] = jnp.dot(x2, wq_ref[...], preferred_element_type=jnp.float32)
    k_ref[...] = jnp.dot(x2, wk_ref[...], preferred_element_type=jnp.float32)
    v_ref[...] = jnp.dot(x2, wv_ref[...], preferred_element_type=jnp.float32)


def node_update(x, num, den, wo, g1, b1, w1, bb1, w2, bb2, g2, b2, wq, wk, wv):
    return pl.pallas_call(
        _node_update_body,
        out_shape=[jax.ShapeDtypeStruct((N, D), jnp.float32)] * 4,
    )(x, num, den, wo, g1, b1, w1, bb1, w2, bb2, g2, b2, wq, wk, wv)


def _node_final_body(x_ref, num_ref, den_ref, batch_ref, wo_ref, g1_ref,
                     b1_ref, w1_ref, bb1_ref, w2_ref, bb2_ref, g2_ref, b2_ref,
                     mw1_ref, mb1_ref, mw2_ref, mb2_ref,
                     vw1_ref, vb1_ref, vw2_ref, vb2_ref,
                     mu_ref, std_ref):
    gm = _group_mat()
    denb = jnp.dot(den_ref[:, 0:H], gm.T, preferred_element_type=jnp.float32)
    agg = num_ref[...] / (denb + 1e-16)
    t = x_ref[...] + jnp.dot(agg, wo_ref[...],
                             preferred_element_type=jnp.float32)
    x1 = _bn_exact(t, g1_ref[...], b1_ref[...])
    h = jnp.maximum(
        jnp.dot(x1, w1_ref[...], preferred_element_type=jnp.float32)
        + bb1_ref[...], 0.0)
    t = x1 + jnp.dot(h, w2_ref[...],
                     preferred_element_type=jnp.float32) + bb2_ref[...]
    x2 = _bn_exact(t, g2_ref[...], b2_ref[...])
    gi = jax.lax.broadcasted_iota(jnp.int32, (N, G), 1)
    onehot = (batch_ref[...] == gi).astype(jnp.float32)
    pooled = jax.lax.dot_general(
        onehot, x2, (((0,), (0,)), ((), ())),
        preferred_element_type=jnp.float32)
    hm = jnp.maximum(
        jnp.dot(pooled, mw1_ref[...], preferred_element_type=jnp.float32)
        + mb1_ref[...], 0.0)
    mu = jnp.dot(hm, mw2_ref[...],
                 preferred_element_type=jnp.float32) + mb2_ref[...]
    hv = jnp.maximum(
        jnp.dot(pooled, vw1_ref[...], preferred_element_type=jnp.float32)
        + vb1_ref[...], 0.0)
    lv = jnp.dot(hv, vw2_ref[...],
                 preferred_element_type=jnp.float32) + vb2_ref[...]
    mu_ref[...] = mu
    std_ref[...] = jnp.exp(0.5 * lv)


def node_final(x, num, den, batch2d, wo, g1, b1, w1, bb1, w2, bb2, g2, b2,
               mw1, mb1, mw2, mb2, vw1, vb1, vw2, vb2):
    return pl.pallas_call(
        _node_final_body,
        out_shape=[jax.ShapeDtypeStruct((G, 1), jnp.float32)] * 2,
    )(x, num, den, batch2d, wo, g1, b1, w1, bb1, w2, bb2, g2, b2,
      mw1, mb1, mw2, mb2, vw1, vb1, vw2, vb2)


def _stats_to_affine(st, cnt, g, b):
    s, ss = st[0], st[1]
    m = s / cnt
    v = ss / cnt - m * m
    sc = g * jax.lax.rsqrt(v + 1e-5)
    return jnp.stack([sc, b - m * sc])


def kernel(x_cat, x_cont, edge_index, edge_attr, pe, batch, params):
    p = params
    src, dst = edge_index[0], edge_index[1]

    xe = jnp.concatenate([p['node_emb'][i][x_cat[:, i]] for i in range(3)],
                         axis=-1)
    ee = jnp.concatenate([p['edge_emb'][i][edge_attr[:, i]] for i in range(2)],
                         axis=-1)

    xf = jnp.concatenate([xe, x_cont], axis=-1)
    x, q, k, v = node_prologue(
        xf, pe, p['node_lin_W'], p['node_lin_b'][None], p['pe_W'],
        p['Wq'][0], p['Wk'][0], p['Wv'][0])

    t2 = edge_init(ee, p['edge_lin_W'], p['edge_lin_b'][None])
    aff = jnp.concatenate([jnp.ones((1, D), jnp.float32),
                           jnp.zeros((1, D), jnp.float32)])

    for l in range(L):
        eprm = epr_kernel(t2, aff, p['We'][l])
        qd = jnp.take(q, dst, axis=0)
        ks = jnp.take(k, src, axis=0)
        vs = jnp.take(v, src, axis=0)
        exv, ex16, t1, st1 = score_fuse(qd, ks, vs, eprm, t2, aff, p['Woe'][l])
        num = jax.ops.segment_sum(exv, dst, num_segments=N)
        den = jax.ops.segment_sum(ex16, dst, num_segments=N)
        aff1 = _stats_to_affine(st1, float(E), p['ebn1_g'][l], p['ebn1_b'][l])
        if l < L - 1:
            t2, st2 = edge_ffn(t1, aff1, p['We1'][l], p['eb1'][l][None],
                               p['We2'][l], p['eb2'][l][None])
            aff = _stats_to_affine(st2, float(E), p['ebn2_g'][l],
                                   p['ebn2_b'][l])
        if l < L - 1:
            x, q, k, v = node_update(
                x, num, den, p['Wo'][l], p['bn1_g'][l][None],
                p['bn1_b'][l][None], p['W1'][l], p['b1'][l][None],
                p['W2'][l], p['b2'][l][None], p['bn2_g'][l][None],
                p['bn2_b'][l][None], p['Wq'][l + 1], p['Wk'][l + 1],
                p['Wv'][l + 1])
        else:
            mu, std = node_final(
                x, num, den, batch[:, None], p['Wo'][l], p['bn1_g'][l][None],
                p['bn1_b'][l][None], p['W1'][l], p['b1'][l][None],
                p['W2'][l], p['b2'][l][None], p['bn2_g'][l][None],
                p['bn2_b'][l][None],
                p['mW1'], p['mb1'][None], p['mW2'], p['mb2'][None],
                p['vW1'], p['vb1'][None], p['vW2'], p['vb2'][None])
    return (mu, std)


# speedup vs baseline: 28.1321x; 2.9667x over previous
import functools
import jax
import jax.numpy as jnp
from jax import lax
from jax.experimental import pallas as pl
from jax.experimental.pallas import tpu as pltpu, tpu_sc as plsc

N, E, G = 10000, 320000, 128
H, DH, D, L = 8, 16, 128, 4
BE = 2000
SCALE = 0.25

NC, NS = 2, 16
NW = NC * NS
CB = 128
NCH = E // CB
SC_ITERS = -(-NCH // NW)
NPT = N // NS
NPT_A = 624
CBN = 80
NCHN = N // CBN
SCN_ITERS = -(-NCHN // NW)

_sc_mesh = plsc.VectorSubcoreMesh(core_axis_name="c", subcore_axis_name="s")


def _wid():
    return lax.axis_index("s") * NC + lax.axis_index("c")


def _group_mat(dtype=jnp.float32):
    d_i = jax.lax.broadcasted_iota(jnp.int32, (D, H), 0)
    h_i = jax.lax.broadcasted_iota(jnp.int32, (D, H), 1)
    return (d_i // DH == h_i).astype(dtype)


def _sc_qkv_gather_body(q_hbm, k_hbm, v_hbm, src_hbm, dst_hbm,
                        qd_hbm, ks_hbm, vs_hbm,
                        srcb, dstb, qb, kb, vb, sem):
    w = _wid()

    def step(i, _):
        t = w + NW * i

        @pl.when(t < NCH)
        def _():
            sl = pl.ds(t * CB, CB)
            pltpu.sync_copy(src_hbm.at[sl], srcb)
            pltpu.sync_copy(dst_hbm.at[sl], dstb)
            d1 = pltpu.async_copy(q_hbm.at[dstb], qb, sem)
            d2 = pltpu.async_copy(k_hbm.at[srcb], kb, sem)
            d3 = pltpu.async_copy(v_hbm.at[srcb], vb, sem)
            d1.wait()
            d2.wait()
            d3.wait()
            pltpu.sync_copy(qb, qd_hbm.at[sl])
            pltpu.sync_copy(kb, ks_hbm.at[sl])
            pltpu.sync_copy(vb, vs_hbm.at[sl])

        return _

    lax.fori_loop(0, SC_ITERS, step, None)


@functools.partial(
    pl.kernel,
    out_type=[jax.ShapeDtypeStruct((E, D), jnp.float32)] * 3,
    mesh=_sc_mesh,
    scratch_types=[
        pltpu.VMEM((CB,), jnp.int32),
        pltpu.VMEM((CB,), jnp.int32),
        pltpu.VMEM((CB, D), jnp.float32),
        pltpu.VMEM((CB, D), jnp.float32),
        pltpu.VMEM((CB, D), jnp.float32),
        pltpu.SemaphoreType.DMA,
    ],
)
def sc_qkv_gather(*refs):
    _sc_qkv_gather_body(*refs)


_NZC = N // CB
_NZT = N - _NZC * CB


def _spmem_zero(s, acc_sh, zbuf):
    def zstep(i, _):
        m = s + NS * i

        @pl.when(m < _NZC)
        def _():
            pltpu.sync_copy(zbuf, acc_sh.at[pl.ds(m * CB, CB)])

        @pl.when(m == _NZC)
        def _():
            pltpu.sync_copy(zbuf.at[pl.ds(0, _NZT)],
                            acc_sh.at[pl.ds(_NZC * CB, _NZT)])

        return _

    lax.fori_loop(0, -(-(_NZC + 1) // NS), zstep, None)


def _spmem_dump(c, s, acc_sh, out_hbm, vbuf):
    def dstep(i, _):
        m = s + NS * i

        @pl.when(m < _NZC)
        def _():
            sl = pl.ds(m * CB, CB)
            pltpu.sync_copy(acc_sh.at[sl], vbuf)
            pltpu.sync_copy(vbuf, out_hbm.at[c, sl])

        @pl.when(m == _NZC)
        def _():
            tl = pl.ds(_NZC * CB, _NZT)
            pltpu.sync_copy(acc_sh.at[tl], vbuf.at[pl.ds(0, _NZT)])
            pltpu.sync_copy(vbuf.at[pl.ds(0, _NZT)], out_hbm.at[c, tl])

        return _

    lax.fori_loop(0, -(-(_NZC + 1) // NS), dstep, None)


def _sc_scatter_body(exv_hbm, exb_hbm, dst_hbm,
                     num2_hbm, den2_hbm,
                     dstb, evb, zbuf, acc_sh):
    c = lax.axis_index("c")
    s = lax.axis_index("s")

    def fill_zero(r, _):
        for j in range(D // 16):
            zbuf[r, pl.ds(j * 16, 16)] = jnp.zeros((16,), jnp.float32)
        return _

    lax.fori_loop(0, CB, fill_zero, None)
    nch_core = NCH // NC
    n_it = -(-nch_core // NS)

    for src_hbm, out_hbm in ((exv_hbm, num2_hbm), (exb_hbm, den2_hbm)):
        _spmem_zero(s, acc_sh, zbuf)
        plsc.subcore_barrier()

        def step(i, _):
            tt = s + NS * i

            @pl.when(tt < nch_core)
            def _():
                sl = pl.ds((c * nch_core + tt) * CB, CB)
                pltpu.sync_copy(dst_hbm.at[sl], dstb)
                pltpu.sync_copy(src_hbm.at[sl], evb)
                pltpu.sync_copy(evb, acc_sh.at[dstb], add=True)

            return _

        lax.fori_loop(0, n_it, step, None)
        plsc.subcore_barrier()
        _spmem_dump(c, s, acc_sh, out_hbm, evb)
        plsc.subcore_barrier()


@functools.partial(
    pl.kernel,
    out_type=[jax.ShapeDtypeStruct((NC, N, D), jnp.float32),
              jax.ShapeDtypeStruct((NC, N, D), jnp.float32)],
    mesh=_sc_mesh,
    scratch_types=[
        pltpu.VMEM((CB,), jnp.int32),
        pltpu.VMEM((CB, D), jnp.float32),
        pltpu.VMEM((CB, D), jnp.float32),
        pltpu.VMEM_SHARED((N, D), jnp.float32),
    ],
)
def sc_scatter_acc(*refs):
    _sc_scatter_body(*refs)


def _sc_embed_body(t0_hbm, t1_hbm, ea0_hbm, ea1_hbm,
                   e0_hbm,
                   eib0, eib1, b0, b1, sem):
    w = _wid()

    def estep(i, _):
        t = w + NW * i

        @pl.when(t < NCH)
        def _():
            sl = pl.ds(t * CB, CB)
            pltpu.sync_copy(ea0_hbm.at[sl], eib0)
            pltpu.sync_copy(ea1_hbm.at[sl], eib1)
            d0 = pltpu.async_copy(t0_hbm.at[eib0], b0, sem)
            d1 = pltpu.async_copy(t1_hbm.at[eib1], b1, sem)
            d0.wait()
            d1.wait()

            def add_row(r, _):
                for j in range(D // 16):
                    cs = pl.ds(j * 16, 16)
                    b0[r, cs] = b0[r, cs] + b1[r, cs]
                return _

            lax.fori_loop(0, CB, add_row, None)
            pltpu.sync_copy(b0, e0_hbm.at[sl])

        return _

    lax.fori_loop(0, SC_ITERS, estep, None)


@functools.partial(
    pl.kernel,
    out_type=jax.ShapeDtypeStruct((E, D), jnp.float32),
    mesh=_sc_mesh,
    scratch_types=[
        pltpu.VMEM((CB,), jnp.int32),
        pltpu.VMEM((CB,), jnp.int32),
        pltpu.VMEM((CB, D), jnp.float32),
        pltpu.VMEM((CB, D), jnp.float32),
        pltpu.SemaphoreType.DMA,
    ],
)
def sc_embed(*refs):
    _sc_embed_body(*refs)


def _edge_tables_body(ee0_ref, ee1_ref, w_ref, b_ref, t0_ref, t1_ref):
    t0_ref[...] = jnp.dot(ee0_ref[...], w_ref[0:8, :],
                          preferred_element_type=jnp.float32) + 0.5 * b_ref[...]
    t1_ref[...] = jnp.dot(ee1_ref[...], w_ref[8:16, :],
                          preferred_element_type=jnp.float32) + 0.5 * b_ref[...]


def edge_tables(ee0, ee1, w, b):
    return pl.pallas_call(
        _edge_tables_body,
        out_shape=[jax.ShapeDtypeStruct((1000, D), jnp.float32)] * 2,
    )(ee0, ee1, w, b)


def _epr_body(t2_ref, aff_ref, w_ref, out_ref):
    el = t2_ref[...] * aff_ref[0:1, :] + aff_ref[1:2, :]
    out_ref[...] = jnp.dot(el, w_ref[...], preferred_element_type=jnp.float32)


def epr_kernel(t2, aff, w):
    return pl.pallas_call(
        _epr_body,
        grid=(E // BE,),
        in_specs=[
            pl.BlockSpec((BE, D), lambda i: (i, 0)),
            pl.BlockSpec((2, D), lambda i: (0, 0)),
            pl.BlockSpec((D, D), lambda i: (0, 0)),
        ],
        out_specs=pl.BlockSpec((BE, D), lambda i: (i, 0)),
        out_shape=jax.ShapeDtypeStruct((E, D), jnp.float32),
    )(t2, aff, w)


def _score_fuse_body(qd_ref, ks_ref, vs_ref, epr_ref, t2_ref, aff_ref, woe_ref,
                     exv_ref, exb_ref, t1_ref, st_ref):
    qk = qd_ref[...] * ks_ref[...] * epr_ref[...] * SCALE
    gm = _group_mat()
    score = jnp.dot(qk, gm, preferred_element_type=jnp.float32)
    ex = jnp.exp(score)
    exb = jnp.dot(ex, gm.T, preferred_element_type=jnp.float32)
    exv_ref[...] = exb * vs_ref[...]
    exb_ref[...] = exb
    el = t2_ref[...] * aff_ref[0:1, :] + aff_ref[1:2, :]
    t1 = el + jnp.dot(qk, woe_ref[...], preferred_element_type=jnp.float32)
    t1_ref[...] = t1
    s = jnp.sum(t1, axis=0, keepdims=True)
    ss = jnp.sum(t1 * t1, axis=0, keepdims=True)
    blk = jnp.concatenate(
        [s, ss, jnp.zeros((6, D), jnp.float32)], axis=0)

    @pl.when(pl.program_id(0) == 0)
    def _():
        st_ref[...] = jnp.zeros_like(st_ref)

    st_ref[...] += blk


def score_fuse(qd, ks, vs, epr, t2, aff, woe):
    return pl.pallas_call(
        _score_fuse_body,
        grid=(E // BE,),
        in_specs=[
            pl.BlockSpec((BE, D), lambda i: (i, 0)),
            pl.BlockSpec((BE, D), lambda i: (i, 0)),
            pl.BlockSpec((BE, D), lambda i: (i, 0)),
            pl.BlockSpec((BE, D), lambda i: (i, 0)),
            pl.BlockSpec((BE, D), lambda i: (i, 0)),
            pl.BlockSpec((2, D), lambda i: (0, 0)),
            pl.BlockSpec((D, D), lambda i: (0, 0)),
        ],
        out_specs=[
            pl.BlockSpec((BE, D), lambda i: (i, 0)),
            pl.BlockSpec((BE, D), lambda i: (i, 0)),
            pl.BlockSpec((BE, D), lambda i: (i, 0)),
            pl.BlockSpec((8, D), lambda i: (0, 0)),
        ],
        out_shape=[
            jax.ShapeDtypeStruct((E, D), jnp.float32),
            jax.ShapeDtypeStruct((E, D), jnp.float32),
            jax.ShapeDtypeStruct((E, D), jnp.float32),
            jax.ShapeDtypeStruct((8, D), jnp.float32),
        ],
    )(qd, ks, vs, epr, t2, aff, woe)


def _edge_ffn_body(t1_ref, aff_ref, w1_ref, b1_ref, w2_ref, b2_ref,
                   t2_ref, st_ref):
    ep = t1_ref[...] * aff_ref[0:1, :] + aff_ref[1:2, :]
    he = jnp.maximum(
        jnp.dot(ep, w1_ref[...], preferred_element_type=jnp.float32)
        + b1_ref[...], 0.0)
    t2 = ep + jnp.dot(he, w2_ref[...],
                      preferred_element_type=jnp.float32) + b2_ref[...]
    t2_ref[...] = t2
    s = jnp.sum(t2, axis=0, keepdims=True)
    ss = jnp.sum(t2 * t2, axis=0, keepdims=True)
    blk = jnp.concatenate([s, ss, jnp.zeros((6, D), jnp.float32)], axis=0)

    @pl.when(pl.program_id(0) == 0)
    def _():
        st_ref[...] = jnp.zeros_like(st_ref)

    st_ref[...] += blk


def edge_ffn(t1, aff, w1, b1, w2, b2):
    return pl.pallas_call(
        _edge_ffn_body,
        grid=(E // BE,),
        in_specs=[
            pl.BlockSpec((BE, D), lambda i: (i, 0)),
            pl.BlockSpec((2, D), lambda i: (0, 0)),
            pl.BlockSpec((D, 2 * D), lambda i: (0, 0)),
            pl.BlockSpec((1, 2 * D), lambda i: (0, 0)),
            pl.BlockSpec((2 * D, D), lambda i: (0, 0)),
            pl.BlockSpec((1, D), lambda i: (0, 0)),
        ],
        out_specs=[
            pl.BlockSpec((BE, D), lambda i: (i, 0)),
            pl.BlockSpec((8, D), lambda i: (0, 0)),
        ],
        out_shape=[
            jax.ShapeDtypeStruct((E, D), jnp.float32),
            jax.ShapeDtypeStruct((8, D), jnp.float32),
        ],
    )(t1, aff, w1, b1, w2, b2)


def _bn_exact(t, g, b):
    m = jnp.mean(t, axis=0, keepdims=True)
    v = jnp.mean(t * t, axis=0, keepdims=True) - m * m
    return (t - m) * jax.lax.rsqrt(v + 1e-5) * g + b


def _node_prologue_body(xf_ref, pe_ref, wn_ref, bn_ref, wpe_ref,
                        wq_ref, wk_ref, wv_ref,
                        x_ref, q_ref, k_ref, v_ref):
    x = (jnp.dot(xf_ref[...], wn_ref[...], preferred_element_type=jnp.float32)
         + bn_ref[...]
         + jnp.dot(pe_ref[...], wpe_ref[...],
                   preferred_element_type=jnp.float32))
    x_ref[...] = x
    q_ref[...] = jnp.dot(x, wq_ref[...], preferred_element_type=jnp.float32)
    k_ref[...] = jnp.dot(x, wk_ref[...], preferred_element_type=jnp.float32)
    v_ref[...] = jnp.dot(x, wv_ref[...], preferred_element_type=jnp.float32)


def node_prologue(xf, pe, wn, bn, wpe, wq, wk, wv):
    return pl.pallas_call(
        _node_prologue_body,
        out_shape=[jax.ShapeDtypeStruct((N, D), jnp.float32)] * 4,
    )(xf, pe, wn, bn, wpe, wq, wk, wv)


def _node_update_body(x_ref, num2_ref, den2_ref, wo_ref, g1_ref, b1_ref,
                      w1_ref, bb1_ref, w2_ref, bb2_ref, g2_ref, b2_ref,
                      wq_ref, wk_ref, wv_ref,
                      x_out_ref, q_ref, k_ref, v_ref):
    denb = den2_ref[0] + den2_ref[1]
    agg = (num2_ref[0] + num2_ref[1]) / (denb + 1e-16)
    t = x_ref[...] + jnp.dot(agg, wo_ref[...],
                             preferred_element_type=jnp.float32)
    x1 = _bn_exact(t, g1_ref[...], b1_ref[...])
    h = jnp.maximum(
        jnp.dot(x1, w1_ref[...], preferred_element_type=jnp.float32)
        + bb1_ref[...], 0.0)
    t = x1 + jnp.dot(h, w2_ref[...],
                     preferred_element_type=jnp.float32) + bb2_ref[...]
    x2 = _bn_exact(t, g2_ref[...], b2_ref[...])
    x_out_ref[...] = x2
    q_ref[...] = jnp.dot(x2, wq_ref[...], preferred_element_type=jnp.float32)
    k_ref[...] = jnp.dot(x2, wk_ref[...], preferred_element_type=jnp.float32)
    v_ref[...] = jnp.dot(x2, wv_ref[...], preferred_element_type=jnp.float32)


def node_update(x, num, den, wo, g1, b1, w1, bb1, w2, bb2, g2, b2, wq, wk, wv):
    return pl.pallas_call(
        _node_update_body,
        out_shape=[jax.ShapeDtypeStruct((N, D), jnp.float32)] * 4,
    )(x, num, den, wo, g1, b1, w1, bb1, w2, bb2, g2, b2, wq, wk, wv)


def _node_final_body(x_ref, num2_ref, den2_ref, batch_ref, wo_ref, g1_ref,
                     b1_ref, w1_ref, bb1_ref, w2_ref, bb2_ref, g2_ref, b2_ref,
                     mw1_ref, mb1_ref, mw2_ref, mb2_ref,
                     vw1_ref, vb1_ref, vw2_ref, vb2_ref,
                     mu_ref, std_ref):
    denb = den2_ref[0] + den2_ref[1]
    agg = (num2_ref[0] + num2_ref[1]) / (denb + 1e-16)
    t = x_ref[...] + jnp.dot(agg, wo_ref[...],
                             preferred_element_type=jnp.float32)
    x1 = _bn_exact(t, g1_ref[...], b1_ref[...])
    h = jnp.maximum(
        jnp.dot(x1, w1_ref[...], preferred_element_type=jnp.float32)
        + bb1_ref[...], 0.0)
    t = x1 + jnp.dot(h, w2_ref[...],
                     preferred_element_type=jnp.float32) + bb2_ref[...]
    x2 = _bn_exact(t, g2_ref[...], b2_ref[...])
    gi = jax.lax.broadcasted_iota(jnp.int32, (N, G), 1)
    onehot = (batch_ref[...] == gi).astype(jnp.float32)
    pooled = jax.lax.dot_general(
        onehot, x2, (((0,), (0,)), ((), ())),
        preferred_element_type=jnp.float32)
    hm = jnp.maximum(
        jnp.dot(pooled, mw1_ref[...], preferred_element_type=jnp.float32)
        + mb1_ref[...], 0.0)
    mu = jnp.dot(hm, mw2_ref[...],
                 preferred_element_type=jnp.float32) + mb2_ref[...]
    hv = jnp.maximum(
        jnp.dot(pooled, vw1_ref[...], preferred_element_type=jnp.float32)
        + vb1_ref[...], 0.0)
    lv = jnp.dot(hv, vw2_ref[...],
                 preferred_element_type=jnp.float32) + vb2_ref[...]
    mu_ref[...] = mu
    std_ref[...] = jnp.exp(0.5 * lv)


def node_final(x, num, den, batch2d, wo, g1, b1, w1, bb1, w2, bb2, g2, b2,
               mw1, mb1, mw2, mb2, vw1, vb1, vw2, vb2):
    return pl.pallas_call(
        _node_final_body,
        out_shape=[jax.ShapeDtypeStruct((G, 1), jnp.float32)] * 2,
    )(x, num, den, batch2d, wo, g1, b1, w1, bb1, w2, bb2, g2, b2,
      mw1, mb1, mw2, mb2, vw1, vb1, vw2, vb2)


def _stats_to_affine(st, cnt, g, b):
    s, ss = st[0], st[1]
    m = s / cnt
    v = ss / cnt - m * m
    sc = g * jax.lax.rsqrt(v + 1e-5)
    return jnp.stack([sc, b - m * sc])


def kernel(x_cat, x_cont, edge_index, edge_attr, pe, batch, params):
    p = params
    src = edge_index[0]
    dst = edge_index[1]

    t0t, t1t = edge_tables(p['edge_emb'][0], p['edge_emb'][1],
                           p['edge_lin_W'], p['edge_lin_b'][None])
    t2 = sc_embed(t0t, t1t, edge_attr[:, 0], edge_attr[:, 1])

    xe = jnp.concatenate([p['node_emb'][i][x_cat[:, i]] for i in range(3)],
                         axis=-1)
    xf = jnp.concatenate([xe, x_cont], axis=-1)
    x, q, k, v = node_prologue(
        xf, pe, p['node_lin_W'], p['node_lin_b'][None], p['pe_W'],
        p['Wq'][0], p['Wk'][0], p['Wv'][0])

    aff = jnp.concatenate([jnp.ones((1, D), jnp.float32),
                           jnp.zeros((1, D), jnp.float32)])

    for l in range(L):
        eprm = epr_kernel(t2, aff, p['We'][l])
        qd, ks, vs = sc_qkv_gather(q, k, v, src, dst)
        exv, exb, t1, st1 = score_fuse(qd, ks, vs, eprm, t2, aff, p['Woe'][l])
        num2, den2 = sc_scatter_acc(exv, exb, dst)
        aff1 = _stats_to_affine(st1, float(E), p['ebn1_g'][l], p['ebn1_b'][l])
        if l < L - 1:
            t2, st2 = edge_ffn(t1, aff1, p['We1'][l], p['eb1'][l][None],
                               p['We2'][l], p['eb2'][l][None])
            aff = _stats_to_affine(st2, float(E), p['ebn2_g'][l],
                                   p['ebn2_b'][l])
        if l < L - 1:
            x, q, k, v = node_update(
                x, num2, den2, p['Wo'][l], p['bn1_g'][l][None],
                p['bn1_b'][l][None], p['W1'][l], p['b1'][l][None],
                p['W2'][l], p['b2'][l][None], p['bn2_g'][l][None],
                p['bn2_b'][l][None], p['Wq'][l + 1], p['Wk'][l + 1],
                p['Wv'][l + 1])
        else:
            mu, std = node_final(
                x, num2, den2, batch[:, None], p['Wo'][l], p['bn1_g'][l][None],
                p['bn1_b'][l][None], p['W1'][l], p['b1'][l][None],
                p['W2'][l], p['b2'][l][None], p['bn2_g'][l][None],
                p['bn2_b'][l][None],
                p['mW1'], p['mb1'][None], p['mW2'], p['mb2'][None],
                p['vW1'], p['vb1'][None], p['vW2'], p['vb2'][None])
    return (mu, std)


# speedup vs baseline: 28.9452x; 1.0289x over previous
import functools
import jax
import jax.numpy as jnp
from jax import lax
from jax.experimental import pallas as pl
from jax.experimental.pallas import tpu as pltpu, tpu_sc as plsc

N, E, G = 10000, 320000, 128
H, DH, D, L = 8, 16, 128, 4
BE = 2000
SCALE = 0.25

NC, NS = 2, 16
NW = NC * NS
CB = 128
NCH = E // CB
SC_ITERS = -(-NCH // NW)
NPT = N // NS
NPT_A = 624
CBN = 80
NCHN = N // CBN
SCN_ITERS = -(-NCHN // NW)

_sc_mesh = plsc.VectorSubcoreMesh(core_axis_name="c", subcore_axis_name="s")


def _wid():
    return lax.axis_index("s") * NC + lax.axis_index("c")


def _group_mat(dtype=jnp.float32):
    d_i = jax.lax.broadcasted_iota(jnp.int32, (D, H), 0)
    h_i = jax.lax.broadcasted_iota(jnp.int32, (D, H), 1)
    return (d_i // DH == h_i).astype(dtype)


def _sc_qkv_gather_body(q_hbm, k_hbm, src_hbm, dst_hbm,
                        qd_hbm, ks_hbm,
                        srcb, dstb, qb, kb, sem):
    w = _wid()

    def step(i, _):
        t = w + NW * i

        @pl.when(t < NCH)
        def _():
            sl = pl.ds(t * CB, CB)
            pltpu.sync_copy(src_hbm.at[sl], srcb)
            pltpu.sync_copy(dst_hbm.at[sl], dstb)
            d1 = pltpu.async_copy(q_hbm.at[dstb], qb, sem)
            d2 = pltpu.async_copy(k_hbm.at[srcb], kb, sem)
            d1.wait()
            d2.wait()
            pltpu.sync_copy(qb, qd_hbm.at[sl])
            pltpu.sync_copy(kb, ks_hbm.at[sl])

        return _

    lax.fori_loop(0, SC_ITERS, step, None)


@functools.partial(
    pl.kernel,
    out_type=[jax.ShapeDtypeStruct((E, D), jnp.float32)] * 2,
    mesh=_sc_mesh,
    scratch_types=[
        pltpu.VMEM((CB,), jnp.int32),
        pltpu.VMEM((CB,), jnp.int32),
        pltpu.VMEM((CB, D), jnp.float32),
        pltpu.VMEM((CB, D), jnp.float32),
        pltpu.SemaphoreType.DMA,
    ],
)
def sc_qkv_gather(*refs):
    _sc_qkv_gather_body(*refs)


_NZC = N // CB
_NZT = N - _NZC * CB


def _spmem_zero(s, acc_sh, zbuf):
    def zstep(i, _):
        m = s + NS * i

        @pl.when(m < _NZC)
        def _():
            pltpu.sync_copy(zbuf, acc_sh.at[pl.ds(m * CB, CB)])

        @pl.when(m == _NZC)
        def _():
            pltpu.sync_copy(zbuf.at[pl.ds(0, _NZT)],
                            acc_sh.at[pl.ds(_NZC * CB, _NZT)])

        return _

    lax.fori_loop(0, -(-(_NZC + 1) // NS), zstep, None)


def _spmem_dump(c, s, acc_sh, out_hbm, vbuf):
    def dstep(i, _):
        m = s + NS * i

        @pl.when(m < _NZC)
        def _():
            sl = pl.ds(m * CB, CB)
            pltpu.sync_copy(acc_sh.at[sl], vbuf)
            pltpu.sync_copy(vbuf, out_hbm.at[c, sl])

        @pl.when(m == _NZC)
        def _():
            tl = pl.ds(_NZC * CB, _NZT)
            pltpu.sync_copy(acc_sh.at[tl], vbuf.at[pl.ds(0, _NZT)])
            pltpu.sync_copy(vbuf.at[pl.ds(0, _NZT)], out_hbm.at[c, tl])

        return _

    lax.fori_loop(0, -(-(_NZC + 1) // NS), dstep, None)


def _sc_scatter_body(exb_hbm, dst_hbm, src_hbm, v_hbm,
                     num2_hbm, den2_hbm,
                     dstb, srcb, exbb, vb, zbuf, acc_sh, sem):
    c = lax.axis_index("c")
    s = lax.axis_index("s")

    def fill_zero(r, _):
        for j in range(D // 16):
            zbuf[r, pl.ds(j * 16, 16)] = jnp.zeros((16,), jnp.float32)
        return _

    lax.fori_loop(0, CB, fill_zero, None)
    nch_core = NCH // NC
    n_it = -(-nch_core // NS)

    _spmem_zero(s, acc_sh, zbuf)
    plsc.subcore_barrier()

    def step1(i, _):
        tt = s + NS * i

        @pl.when(tt < nch_core)
        def _():
            sl = pl.ds((c * nch_core + tt) * CB, CB)
            pltpu.sync_copy(src_hbm.at[sl], srcb)
            dg = pltpu.async_copy(v_hbm.at[srcb], vb, sem)
            pltpu.sync_copy(dst_hbm.at[sl], dstb)
            pltpu.sync_copy(exb_hbm.at[sl], exbb)
            dg.wait()

            def mul_row(r, _):
                for j in range(D // 16):
                    cs = pl.ds(j * 16, 16)
                    vb[r, cs] = vb[r, cs] * exbb[r, cs]
                return _

            lax.fori_loop(0, CB, mul_row, None)
            pltpu.sync_copy(vb, acc_sh.at[dstb], add=True)

        return _

    lax.fori_loop(0, n_it, step1, None)
    plsc.subcore_barrier()
    _spmem_dump(c, s, acc_sh, num2_hbm, vb)
    _spmem_zero(s, acc_sh, zbuf)
    plsc.subcore_barrier()

    def step2(i, _):
        tt = s + NS * i

        @pl.when(tt < nch_core)
        def _():
            sl = pl.ds((c * nch_core + tt) * CB, CB)
            pltpu.sync_copy(dst_hbm.at[sl], dstb)
            pltpu.sync_copy(exb_hbm.at[sl], exbb)
            pltpu.sync_copy(exbb, acc_sh.at[dstb], add=True)

        return _

    lax.fori_loop(0, n_it, step2, None)
    plsc.subcore_barrier()
    _spmem_dump(c, s, acc_sh, den2_hbm, vb)


@functools.partial(
    pl.kernel,
    out_type=[jax.ShapeDtypeStruct((NC, N, D), jnp.float32),
              jax.ShapeDtypeStruct((NC, N, D), jnp.float32)],
    mesh=_sc_mesh,
    scratch_types=[
        pltpu.VMEM((CB,), jnp.int32),
        pltpu.VMEM((CB,), jnp.int32),
        pltpu.VMEM((CB, D), jnp.float32),
        pltpu.VMEM((CB, D), jnp.float32),
        pltpu.VMEM((CB, D), jnp.float32),
        pltpu.VMEM_SHARED((N, D), jnp.float32),
        pltpu.SemaphoreType.DMA,
    ],
)
def sc_scatter_acc(*refs):
    _sc_scatter_body(*refs)


def _sc_embed_body(t0_hbm, t1_hbm, ea0_hbm, ea1_hbm,
                   e0_hbm,
                   eib0, eib1, b0, b1, sem):
    w = _wid()

    def estep(i, _):
        t = w + NW * i

        @pl.when(t < NCH)
        def _():
            sl = pl.ds(t * CB, CB)
            pltpu.sync_copy(ea0_hbm.at[sl], eib0)
            pltpu.sync_copy(ea1_hbm.at[sl], eib1)
            d0 = pltpu.async_copy(t0_hbm.at[eib0], b0, sem)
            d1 = pltpu.async_copy(t1_hbm.at[eib1], b1, sem)
            d0.wait()
            d1.wait()

            def add_row(r, _):
                for j in range(D // 16):
                    cs = pl.ds(j * 16, 16)
                    b0[r, cs] = b0[r, cs] + b1[r, cs]
                return _

            lax.fori_loop(0, CB, add_row, None)
            pltpu.sync_copy(b0, e0_hbm.at[sl])

        return _

    lax.fori_loop(0, SC_ITERS, estep, None)


@functools.partial(
    pl.kernel,
    out_type=jax.ShapeDtypeStruct((E, D), jnp.float32),
    mesh=_sc_mesh,
    scratch_types=[
        pltpu.VMEM((CB,), jnp.int32),
        pltpu.VMEM((CB,), jnp.int32),
        pltpu.VMEM((CB, D), jnp.float32),
        pltpu.VMEM((CB, D), jnp.float32),
        pltpu.SemaphoreType.DMA,
    ],
)
def sc_embed(*refs):
    _sc_embed_body(*refs)


def _edge_tables_body(ee0_ref, ee1_ref, w_ref, b_ref, t0_ref, t1_ref):
    t0_ref[...] = jnp.dot(ee0_ref[...], w_ref[0:8, :],
                          preferred_element_type=jnp.float32) + 0.5 * b_ref[...]
    t1_ref[...] = jnp.dot(ee1_ref[...], w_ref[8:16, :],
                          preferred_element_type=jnp.float32) + 0.5 * b_ref[...]


def edge_tables(ee0, ee1, w, b):
    return pl.pallas_call(
        _edge_tables_body,
        out_shape=[jax.ShapeDtypeStruct((1000, D), jnp.float32)] * 2,
    )(ee0, ee1, w, b)


def _epr_body(t2_ref, aff_ref, w_ref, out_ref):
    el = t2_ref[...] * aff_ref[0:1, :] + aff_ref[1:2, :]
    out_ref[...] = jnp.dot(el, w_ref[...], preferred_element_type=jnp.float32)


def epr_kernel(t2, aff, w):
    return pl.pallas_call(
        _epr_body,
        grid=(E // BE,),
        in_specs=[
            pl.BlockSpec((BE, D), lambda i: (i, 0)),
            pl.BlockSpec((2, D), lambda i: (0, 0)),
            pl.BlockSpec((D, D), lambda i: (0, 0)),
        ],
        out_specs=pl.BlockSpec((BE, D), lambda i: (i, 0)),
        out_shape=jax.ShapeDtypeStruct((E, D), jnp.float32),
    )(t2, aff, w)


def _score_fuse_body(qd_ref, ks_ref, epr_ref, t2_ref, aff_ref, woe_ref,
                     exb_ref, t1_ref, st_ref):
    qk = qd_ref[...] * ks_ref[...] * epr_ref[...] * SCALE
    gm = _group_mat()
    score = jnp.dot(qk, gm, preferred_element_type=jnp.float32)
    ex = jnp.exp(score)
    exb = jnp.dot(ex, gm.T, preferred_element_type=jnp.float32)
    exb_ref[...] = exb
    el = t2_ref[...] * aff_ref[0:1, :] + aff_ref[1:2, :]
    t1 = el + jnp.dot(qk, woe_ref[...], preferred_element_type=jnp.float32)
    t1_ref[...] = t1
    s = jnp.sum(t1, axis=0, keepdims=True)
    ss = jnp.sum(t1 * t1, axis=0, keepdims=True)
    blk = jnp.concatenate(
        [s, ss, jnp.zeros((6, D), jnp.float32)], axis=0)

    @pl.when(pl.program_id(0) == 0)
    def _():
        st_ref[...] = jnp.zeros_like(st_ref)

    st_ref[...] += blk


def score_fuse(qd, ks, epr, t2, aff, woe):
    return pl.pallas_call(
        _score_fuse_body,
        grid=(E // BE,),
        in_specs=[
            pl.BlockSpec((BE, D), lambda i: (i, 0)),
            pl.BlockSpec((BE, D), lambda i: (i, 0)),
            pl.BlockSpec((BE, D), lambda i: (i, 0)),
            pl.BlockSpec((BE, D), lambda i: (i, 0)),
            pl.BlockSpec((2, D), lambda i: (0, 0)),
            pl.BlockSpec((D, D), lambda i: (0, 0)),
        ],
        out_specs=[
            pl.BlockSpec((BE, D), lambda i: (i, 0)),
            pl.BlockSpec((BE, D), lambda i: (i, 0)),
            pl.BlockSpec((8, D), lambda i: (0, 0)),
        ],
        out_shape=[
            jax.ShapeDtypeStruct((E, D), jnp.float32),
            jax.ShapeDtypeStruct((E, D), jnp.float32),
            jax.ShapeDtypeStruct((8, D), jnp.float32),
        ],
    )(qd, ks, epr, t2, aff, woe)


def _edge_ffn_body(t1_ref, aff_ref, w1_ref, b1_ref, w2_ref, b2_ref,
                   t2_ref, st_ref):
    ep = t1_ref[...] * aff_ref[0:1, :] + aff_ref[1:2, :]
    he = jnp.maximum(
        jnp.dot(ep, w1_ref[...], preferred_element_type=jnp.float32)
        + b1_ref[...], 0.0)
    t2 = ep + jnp.dot(he, w2_ref[...],
                      preferred_element_type=jnp.float32) + b2_ref[...]
    t2_ref[...] = t2
    s = jnp.sum(t2, axis=0, keepdims=True)
    ss = jnp.sum(t2 * t2, axis=0, keepdims=True)
    blk = jnp.concatenate([s, ss, jnp.zeros((6, D), jnp.float32)], axis=0)

    @pl.when(pl.program_id(0) == 0)
    def _():
        st_ref[...] = jnp.zeros_like(st_ref)

    st_ref[...] += blk


def edge_ffn(t1, aff, w1, b1, w2, b2):
    return pl.pallas_call(
        _edge_ffn_body,
        grid=(E // BE,),
        in_specs=[
            pl.BlockSpec((BE, D), lambda i: (i, 0)),
            pl.BlockSpec((2, D), lambda i: (0, 0)),
            pl.BlockSpec((D, 2 * D), lambda i: (0, 0)),
            pl.BlockSpec((1, 2 * D), lambda i: (0, 0)),
            pl.BlockSpec((2 * D, D), lambda i: (0, 0)),
            pl.BlockSpec((1, D), lambda i: (0, 0)),
        ],
        out_specs=[
            pl.BlockSpec((BE, D), lambda i: (i, 0)),
            pl.BlockSpec((8, D), lambda i: (0, 0)),
        ],
        out_shape=[
            jax.ShapeDtypeStruct((E, D), jnp.float32),
            jax.ShapeDtypeStruct((8, D), jnp.float32),
        ],
    )(t1, aff, w1, b1, w2, b2)


def _bn_exact(t, g, b):
    m = jnp.mean(t, axis=0, keepdims=True)
    v = jnp.mean(t * t, axis=0, keepdims=True) - m * m
    return (t - m) * jax.lax.rsqrt(v + 1e-5) * g + b


def _node_prologue_body(xf_ref, pe_ref, wn_ref, bn_ref, wpe_ref,
                        wq_ref, wk_ref, wv_ref,
                        x_ref, q_ref, k_ref, v_ref):
    x = (jnp.dot(xf_ref[...], wn_ref[...], preferred_element_type=jnp.float32)
         + bn_ref[...]
         + jnp.dot(pe_ref[...], wpe_ref[...],
                   preferred_element_type=jnp.float32))
    x_ref[...] = x
    q_ref[...] = jnp.dot(x, wq_ref[...], preferred_element_type=jnp.float32)
    k_ref[...] = jnp.dot(x, wk_ref[...], preferred_element_type=jnp.float32)
    v_ref[...] = jnp.dot(x, wv_ref[...], preferred_element_type=jnp.float32)


def node_prologue(xf, pe, wn, bn, wpe, wq, wk, wv):
    return pl.pallas_call(
        _node_prologue_body,
        out_shape=[jax.ShapeDtypeStruct((N, D), jnp.float32)] * 4,
    )(xf, pe, wn, bn, wpe, wq, wk, wv)


def _node_update_body(x_ref, num2_ref, den2_ref, wo_ref, g1_ref, b1_ref,
                      w1_ref, bb1_ref, w2_ref, bb2_ref, g2_ref, b2_ref,
                      wq_ref, wk_ref, wv_ref,
                      x_out_ref, q_ref, k_ref, v_ref):
    denb = den2_ref[0] + den2_ref[1]
    agg = (num2_ref[0] + num2_ref[1]) / (denb + 1e-16)
    t = x_ref[...] + jnp.dot(agg, wo_ref[...],
                             preferred_element_type=jnp.float32)
    x1 = _bn_exact(t, g1_ref[...], b1_ref[...])
    h = jnp.maximum(
        jnp.dot(x1, w1_ref[...], preferred_element_type=jnp.float32)
        + bb1_ref[...], 0.0)
    t = x1 + jnp.dot(h, w2_ref[...],
                     preferred_element_type=jnp.float32) + bb2_ref[...]
    x2 = _bn_exact(t, g2_ref[...], b2_ref[...])
    x_out_ref[...] = x2
    q_ref[...] = jnp.dot(x2, wq_ref[...], preferred_element_type=jnp.float32)
    k_ref[...] = jnp.dot(x2, wk_ref[...], preferred_element_type=jnp.float32)
    v_ref[...] = jnp.dot(x2, wv_ref[...], preferred_element_type=jnp.float32)


def node_update(x, num, den, wo, g1, b1, w1, bb1, w2, bb2, g2, b2, wq, wk, wv):
    return pl.pallas_call(
        _node_update_body,
        out_shape=[jax.ShapeDtypeStruct((N, D), jnp.float32)] * 4,
    )(x, num, den, wo, g1, b1, w1, bb1, w2, bb2, g2, b2, wq, wk, wv)


def _node_final_body(x_ref, num2_ref, den2_ref, batch_ref, wo_ref, g1_ref,
                     b1_ref, w1_ref, bb1_ref, w2_ref, bb2_ref, g2_ref, b2_ref,
                     mw1_ref, mb1_ref, mw2_ref, mb2_ref,
                     vw1_ref, vb1_ref, vw2_ref, vb2_ref,
                     mu_ref, std_ref):
    denb = den2_ref[0] + den2_ref[1]
    agg = (num2_ref[0] + num2_ref[1]) / (denb + 1e-16)
    t = x_ref[...] + jnp.dot(agg, wo_ref[...],
                             preferred_element_type=jnp.float32)
    x1 = _bn_exact(t, g1_ref[...], b1_ref[...])
    h = jnp.maximum(
        jnp.dot(x1, w1_ref[...], preferred_element_type=jnp.float32)
        + bb1_ref[...], 0.0)
    t = x1 + jnp.dot(h, w2_ref[...],
                     preferred_element_type=jnp.float32) + bb2_ref[...]
    x2 = _bn_exact(t, g2_ref[...], b2_ref[...])
    gi = jax.lax.broadcasted_iota(jnp.int32, (N, G), 1)
    onehot = (batch_ref[...] == gi).astype(jnp.float32)
    pooled = jax.lax.dot_general(
        onehot, x2, (((0,), (0,)), ((), ())),
        preferred_element_type=jnp.float32)
    hm = jnp.maximum(
        jnp.dot(pooled, mw1_ref[...], preferred_element_type=jnp.float32)
        + mb1_ref[...], 0.0)
    mu = jnp.dot(hm, mw2_ref[...],
                 preferred_element_type=jnp.float32) + mb2_ref[...]
    hv = jnp.maximum(
        jnp.dot(pooled, vw1_ref[...], preferred_element_type=jnp.float32)
        + vb1_ref[...], 0.0)
    lv = jnp.dot(hv, vw2_ref[...],
                 preferred_element_type=jnp.float32) + vb2_ref[...]
    mu_ref[...] = mu
    std_ref[...] = jnp.exp(0.5 * lv)


def node_final(x, num, den, batch2d, wo, g1, b1, w1, bb1, w2, bb2, g2, b2,
               mw1, mb1, mw2, mb2, vw1, vb1, vw2, vb2):
    return pl.pallas_call(
        _node_final_body,
        out_shape=[jax.ShapeDtypeStruct((G, 1), jnp.float32)] * 2,
    )(x, num, den, batch2d, wo, g1, b1, w1, bb1, w2, bb2, g2, b2,
      mw1, mb1, mw2, mb2, vw1, vb1, vw2, vb2)


def _stats_to_affine(st, cnt, g, b):
    s, ss = st[0], st[1]
    m = s / cnt
    v = ss / cnt - m * m
    sc = g * jax.lax.rsqrt(v + 1e-5)
    return jnp.stack([sc, b - m * sc])


def kernel(x_cat, x_cont, edge_index, edge_attr, pe, batch, params):
    p = params
    src = edge_index[0]
    dst = edge_index[1]

    t0t, t1t = edge_tables(p['edge_emb'][0], p['edge_emb'][1],
                           p['edge_lin_W'], p['edge_lin_b'][None])
    t2 = sc_embed(t0t, t1t, edge_attr[:, 0], edge_attr[:, 1])

    xe = jnp.concatenate([p['node_emb'][i][x_cat[:, i]] for i in range(3)],
                         axis=-1)
    xf = jnp.concatenate([xe, x_cont], axis=-1)
    x, q, k, v = node_prologue(
        xf, pe, p['node_lin_W'], p['node_lin_b'][None], p['pe_W'],
        p['Wq'][0], p['Wk'][0], p['Wv'][0])

    aff = jnp.concatenate([jnp.ones((1, D), jnp.float32),
                           jnp.zeros((1, D), jnp.float32)])

    for l in range(L):
        eprm = epr_kernel(t2, aff, p['We'][l])
        qd, ks = sc_qkv_gather(q, k, src, dst)
        exb, t1, st1 = score_fuse(qd, ks, eprm, t2, aff, p['Woe'][l])
        num2, den2 = sc_scatter_acc(exb, dst, src, v)
        aff1 = _stats_to_affine(st1, float(E), p['ebn1_g'][l], p['ebn1_b'][l])
        if l < L - 1:
            t2, st2 = edge_ffn(t1, aff1, p['We1'][l], p['eb1'][l][None],
                               p['We2'][l], p['eb2'][l][None])
            aff = _stats_to_affine(st2, float(E), p['ebn2_g'][l],
                                   p['ebn2_b'][l])
        if l < L - 1:
            x, q, k, v = node_update(
                x, num2, den2, p['Wo'][l], p['bn1_g'][l][None],
                p['bn1_b'][l][None], p['W1'][l], p['b1'][l][None],
                p['W2'][l], p['b2'][l][None], p['bn2_g'][l][None],
                p['bn2_b'][l][None], p['Wq'][l + 1], p['Wk'][l + 1],
                p['Wv'][l + 1])
        else:
            mu, std = node_final(
                x, num2, den2, batch[:, None], p['Wo'][l], p['bn1_g'][l][None],
                p['bn1_b'][l][None], p['W1'][l], p['b1'][l][None],
                p['W2'][l], p['b2'][l][None], p['bn2_g'][l][None],
                p['bn2_b'][l][None],
                p['mW1'], p['mb1'][None], p['mW2'], p['mb2'][None],
                p['vW1'], p['vb1'][None], p['vW2'], p['vb2'][None])
    return (mu, std)


# speedup vs baseline: 30.3853x; 1.0498x over previous
import functools
import jax
import jax.numpy as jnp
from jax import lax
from jax.experimental import pallas as pl
from jax.experimental.pallas import tpu as pltpu, tpu_sc as plsc

N, E, G = 10000, 320000, 128
H, DH, D, L = 8, 16, 128, 4
BE = 2000
SCALE = 0.25

NC, NS = 2, 16
NW = NC * NS
CB = 128
NCH = E // CB
SC_ITERS = -(-NCH // NW)
NPT = N // NS
NPT_A = 624
CBN = 80
NCHN = N // CBN
SCN_ITERS = -(-NCHN // NW)

_sc_mesh = plsc.VectorSubcoreMesh(core_axis_name="c", subcore_axis_name="s")


def _wid():
    return lax.axis_index("s") * NC + lax.axis_index("c")


def _group_mat(dtype=jnp.float32):
    d_i = jax.lax.broadcasted_iota(jnp.int32, (D, H), 0)
    h_i = jax.lax.broadcasted_iota(jnp.int32, (D, H), 1)
    return (d_i // DH == h_i).astype(dtype)


def _sc_qkv_gather_body(q_hbm, k_hbm, src_hbm, dst_hbm,
                        qk_hbm,
                        srcb, dstb, qb, kb, sem):
    w = _wid()

    def step(i, _):
        t = w + NW * i

        @pl.when(t < NCH)
        def _():
            sl = pl.ds(t * CB, CB)
            pltpu.sync_copy(src_hbm.at[sl], srcb)
            pltpu.sync_copy(dst_hbm.at[sl], dstb)
            d1 = pltpu.async_copy(q_hbm.at[dstb], qb, sem)
            d2 = pltpu.async_copy(k_hbm.at[srcb], kb, sem)
            d1.wait()
            d2.wait()

            def mul_row(r, _):
                for j in range(D // 16):
                    cs = pl.ds(j * 16, 16)
                    qb[r, cs] = qb[r, cs] * kb[r, cs]
                return _

            lax.fori_loop(0, CB, mul_row, None)
            pltpu.sync_copy(qb, qk_hbm.at[sl])

        return _

    lax.fori_loop(0, SC_ITERS, step, None)


@functools.partial(
    pl.kernel,
    out_type=jax.ShapeDtypeStruct((E, D), jnp.float32),
    mesh=_sc_mesh,
    scratch_types=[
        pltpu.VMEM((CB,), jnp.int32),
        pltpu.VMEM((CB,), jnp.int32),
        pltpu.VMEM((CB, D), jnp.float32),
        pltpu.VMEM((CB, D), jnp.float32),
        pltpu.SemaphoreType.DMA,
    ],
)
def sc_qkv_gather(*refs):
    _sc_qkv_gather_body(*refs)


_NZC = N // CB
_NZT = N - _NZC * CB


def _spmem_zero(s, acc_sh, zbuf):
    def zstep(i, _):
        m = s + NS * i

        @pl.when(m < _NZC)
        def _():
            pltpu.sync_copy(zbuf, acc_sh.at[pl.ds(m * CB, CB)])

        @pl.when(m == _NZC)
        def _():
            pltpu.sync_copy(zbuf.at[pl.ds(0, _NZT)],
                            acc_sh.at[pl.ds(_NZC * CB, _NZT)])

        return _

    lax.fori_loop(0, -(-(_NZC + 1) // NS), zstep, None)


def _spmem_dump(c, s, acc_sh, out_hbm, vbuf):
    def dstep(i, _):
        m = s + NS * i

        @pl.when(m < _NZC)
        def _():
            sl = pl.ds(m * CB, CB)
            pltpu.sync_copy(acc_sh.at[sl], vbuf)
            pltpu.sync_copy(vbuf, out_hbm.at[c, sl])

        @pl.when(m == _NZC)
        def _():
            tl = pl.ds(_NZC * CB, _NZT)
            pltpu.sync_copy(acc_sh.at[tl], vbuf.at[pl.ds(0, _NZT)])
            pltpu.sync_copy(vbuf.at[pl.ds(0, _NZT)], out_hbm.at[c, tl])

        return _

    lax.fori_loop(0, -(-(_NZC + 1) // NS), dstep, None)


def _sc_scatter_body(exb_hbm, dst_hbm, src_hbm, v_hbm,
                     num2_hbm, den2_hbm,
                     dstb, srcb, exbb, vb, zbuf, acc_sh, sem):
    c = lax.axis_index("c")
    s = lax.axis_index("s")

    def fill_zero(r, _):
        for j in range(D // 16):
            zbuf[r, pl.ds(j * 16, 16)] = jnp.zeros((16,), jnp.float32)
        return _

    lax.fori_loop(0, CB, fill_zero, None)
    nch_core = NCH // NC
    n_it = -(-nch_core // NS)

    _spmem_zero(s, acc_sh, zbuf)
    plsc.subcore_barrier()

    def step1(i, _):
        tt = s + NS * i

        @pl.when(tt < nch_core)
        def _():
            sl = pl.ds((c * nch_core + tt) * CB, CB)
            pltpu.sync_copy(src_hbm.at[sl], srcb)
            dg = pltpu.async_copy(v_hbm.at[srcb], vb, sem)
            pltpu.sync_copy(dst_hbm.at[sl], dstb)
            pltpu.sync_copy(exb_hbm.at[sl], exbb)
            dg.wait()

            def mul_row(r, _):
                for j in range(D // 16):
                    cs = pl.ds(j * 16, 16)
                    vb[r, cs] = vb[r, cs] * exbb[r, cs]
                return _

            lax.fori_loop(0, CB, mul_row, None)
            pltpu.sync_copy(vb, acc_sh.at[dstb], add=True)

        return _

    lax.fori_loop(0, n_it, step1, None)
    plsc.subcore_barrier()
    _spmem_dump(c, s, acc_sh, num2_hbm, vb)
    _spmem_zero(s, acc_sh, zbuf)
    plsc.subcore_barrier()

    def step2(i, _):
        tt = s + NS * i

        @pl.when(tt < nch_core)
        def _():
            sl = pl.ds((c * nch_core + tt) * CB, CB)
            pltpu.sync_copy(dst_hbm.at[sl], dstb)
            pltpu.sync_copy(exb_hbm.at[sl], exbb)
            pltpu.sync_copy(exbb, acc_sh.at[dstb], add=True)

        return _

    lax.fori_loop(0, n_it, step2, None)
    plsc.subcore_barrier()
    _spmem_dump(c, s, acc_sh, den2_hbm, vb)


@functools.partial(
    pl.kernel,
    out_type=[jax.ShapeDtypeStruct((NC, N, D), jnp.float32),
              jax.ShapeDtypeStruct((NC, N, D), jnp.float32)],
    mesh=_sc_mesh,
    scratch_types=[
        pltpu.VMEM((CB,), jnp.int32),
        pltpu.VMEM((CB,), jnp.int32),
        pltpu.VMEM((CB, D), jnp.float32),
        pltpu.VMEM((CB, D), jnp.float32),
        pltpu.VMEM((CB, D), jnp.float32),
        pltpu.VMEM_SHARED((N, D), jnp.float32),
        pltpu.SemaphoreType.DMA,
    ],
)
def sc_scatter_acc(*refs):
    _sc_scatter_body(*refs)


def _sc_embed_body(t0_hbm, t1_hbm, ea0_hbm, ea1_hbm,
                   e0_hbm,
                   eib0, eib1, b0, b1, sem):
    w = _wid()

    def estep(i, _):
        t = w + NW * i

        @pl.when(t < NCH)
        def _():
            sl = pl.ds(t * CB, CB)
            pltpu.sync_copy(ea0_hbm.at[sl], eib0)
            pltpu.sync_copy(ea1_hbm.at[sl], eib1)
            d0 = pltpu.async_copy(t0_hbm.at[eib0], b0, sem)
            d1 = pltpu.async_copy(t1_hbm.at[eib1], b1, sem)
            d0.wait()
            d1.wait()

            def add_row(r, _):
                for j in range(D // 16):
                    cs = pl.ds(j * 16, 16)
                    b0[r, cs] = b0[r, cs] + b1[r, cs]
                return _

            lax.fori_loop(0, CB, add_row, None)
            pltpu.sync_copy(b0, e0_hbm.at[sl])

        return _

    lax.fori_loop(0, SC_ITERS, estep, None)


@functools.partial(
    pl.kernel,
    out_type=jax.ShapeDtypeStruct((E, D), jnp.float32),
    mesh=_sc_mesh,
    scratch_types=[
        pltpu.VMEM((CB,), jnp.int32),
        pltpu.VMEM((CB,), jnp.int32),
        pltpu.VMEM((CB, D), jnp.float32),
        pltpu.VMEM((CB, D), jnp.float32),
        pltpu.SemaphoreType.DMA,
    ],
)
def sc_embed(*refs):
    _sc_embed_body(*refs)


def _edge_tables_body(ee0_ref, ee1_ref, w_ref, b_ref, t0_ref, t1_ref):
    t0_ref[...] = jnp.dot(ee0_ref[...], w_ref[0:8, :],
                          preferred_element_type=jnp.float32) + 0.5 * b_ref[...]
    t1_ref[...] = jnp.dot(ee1_ref[...], w_ref[8:16, :],
                          preferred_element_type=jnp.float32) + 0.5 * b_ref[...]


def edge_tables(ee0, ee1, w, b):
    return pl.pallas_call(
        _edge_tables_body,
        out_shape=[jax.ShapeDtypeStruct((1000, D), jnp.float32)] * 2,
    )(ee0, ee1, w, b)


def _epr_body(t2_ref, aff_ref, w_ref, out_ref):
    el = t2_ref[...] * aff_ref[0:1, :] + aff_ref[1:2, :]
    out_ref[...] = jnp.dot(el, w_ref[...], preferred_element_type=jnp.float32)


def epr_kernel(t2, aff, w):
    return pl.pallas_call(
        _epr_body,
        grid=(E // BE,),
        in_specs=[
            pl.BlockSpec((BE, D), lambda i: (i, 0)),
            pl.BlockSpec((2, D), lambda i: (0, 0)),
            pl.BlockSpec((D, D), lambda i: (0, 0)),
        ],
        out_specs=pl.BlockSpec((BE, D), lambda i: (i, 0)),
        out_shape=jax.ShapeDtypeStruct((E, D), jnp.float32),
    )(t2, aff, w)


def _score_fuse_body(qkpre_ref, epr_ref, t2_ref, aff_ref, woe_ref,
                     exb_ref, t1_ref, st_ref):
    qk = qkpre_ref[...] * epr_ref[...] * SCALE
    gm = _group_mat()
    score = jnp.dot(qk, gm, preferred_element_type=jnp.float32)
    ex = jnp.exp(score)
    exb = jnp.dot(ex, gm.T, preferred_element_type=jnp.float32)
    exb_ref[...] = exb
    el = t2_ref[...] * aff_ref[0:1, :] + aff_ref[1:2, :]
    t1 = el + jnp.dot(qk, woe_ref[...], preferred_element_type=jnp.float32)
    t1_ref[...] = t1
    s = jnp.sum(t1, axis=0, keepdims=True)
    ss = jnp.sum(t1 * t1, axis=0, keepdims=True)
    blk = jnp.concatenate(
        [s, ss, jnp.zeros((6, D), jnp.float32)], axis=0)

    @pl.when(pl.program_id(0) == 0)
    def _():
        st_ref[...] = jnp.zeros_like(st_ref)

    st_ref[...] += blk


def score_fuse(qkpre, epr, t2, aff, woe):
    return pl.pallas_call(
        _score_fuse_body,
        grid=(E // BE,),
        in_specs=[
            pl.BlockSpec((BE, D), lambda i: (i, 0)),
            pl.BlockSpec((BE, D), lambda i: (i, 0)),
            pl.BlockSpec((BE, D), lambda i: (i, 0)),
            pl.BlockSpec((2, D), lambda i: (0, 0)),
            pl.BlockSpec((D, D), lambda i: (0, 0)),
        ],
        out_specs=[
            pl.BlockSpec((BE, D), lambda i: (i, 0)),
            pl.BlockSpec((BE, D), lambda i: (i, 0)),
            pl.BlockSpec((8, D), lambda i: (0, 0)),
        ],
        out_shape=[
            jax.ShapeDtypeStruct((E, D), jnp.float32),
            jax.ShapeDtypeStruct((E, D), jnp.float32),
            jax.ShapeDtypeStruct((8, D), jnp.float32),
        ],
    )(qkpre, epr, t2, aff, woe)


def _edge_ffn_body(t1_ref, aff_ref, w1_ref, b1_ref, w2_ref, b2_ref,
                   t2_ref, st_ref):
    ep = t1_ref[...] * aff_ref[0:1, :] + aff_ref[1:2, :]
    he = jnp.maximum(
        jnp.dot(ep, w1_ref[...], preferred_element_type=jnp.float32)
        + b1_ref[...], 0.0)
    t2 = ep + jnp.dot(he, w2_ref[...],
                      preferred_element_type=jnp.float32) + b2_ref[...]
    t2_ref[...] = t2
    s = jnp.sum(t2, axis=0, keepdims=True)
    ss = jnp.sum(t2 * t2, axis=0, keepdims=True)
    blk = jnp.concatenate([s, ss, jnp.zeros((6, D), jnp.float32)], axis=0)

    @pl.when(pl.program_id(0) == 0)
    def _():
        st_ref[...] = jnp.zeros_like(st_ref)

    st_ref[...] += blk


def edge_ffn(t1, aff, w1, b1, w2, b2):
    return pl.pallas_call(
        _edge_ffn_body,
        grid=(E // BE,),
        in_specs=[
            pl.BlockSpec((BE, D), lambda i: (i, 0)),
            pl.BlockSpec((2, D), lambda i: (0, 0)),
            pl.BlockSpec((D, 2 * D), lambda i: (0, 0)),
            pl.BlockSpec((1, 2 * D), lambda i: (0, 0)),
            pl.BlockSpec((2 * D, D), lambda i: (0, 0)),
            pl.BlockSpec((1, D), lambda i: (0, 0)),
        ],
        out_specs=[
            pl.BlockSpec((BE, D), lambda i: (i, 0)),
            pl.BlockSpec((8, D), lambda i: (0, 0)),
        ],
        out_shape=[
            jax.ShapeDtypeStruct((E, D), jnp.float32),
            jax.ShapeDtypeStruct((8, D), jnp.float32),
        ],
    )(t1, aff, w1, b1, w2, b2)


def _bn_exact(t, g, b):
    m = jnp.mean(t, axis=0, keepdims=True)
    v = jnp.mean(t * t, axis=0, keepdims=True) - m * m
    return (t - m) * jax.lax.rsqrt(v + 1e-5) * g + b


def _node_prologue_body(xf_ref, pe_ref, wn_ref, bn_ref, wpe_ref,
                        wq_ref, wk_ref, wv_ref,
                        x_ref, q_ref, k_ref, v_ref):
    x = (jnp.dot(xf_ref[...], wn_ref[...], preferred_element_type=jnp.float32)
         + bn_ref[...]
         + jnp.dot(pe_ref[...], wpe_ref[...],
                   preferred_element_type=jnp.float32))
    x_ref[...] = x
    q_ref[...] = jnp.dot(x, wq_ref[...], preferred_element_type=jnp.float32)
    k_ref[...] = jnp.dot(x, wk_ref[...], preferred_element_type=jnp.float32)
    v_ref[...] = jnp.dot(x, wv_ref[...], preferred_element_type=jnp.float32)


def node_prologue(xf, pe, wn, bn, wpe, wq, wk, wv):
    return pl.pallas_call(
        _node_prologue_body,
        out_shape=[jax.ShapeDtypeStruct((N, D), jnp.float32)] * 4,
    )(xf, pe, wn, bn, wpe, wq, wk, wv)


def _node_update_body(x_ref, num2_ref, den2_ref, wo_ref, g1_ref, b1_ref,
                      w1_ref, bb1_ref, w2_ref, bb2_ref, g2_ref, b2_ref,
                      wq_ref, wk_ref, wv_ref,
                      x_out_ref, q_ref, k_ref, v_ref):
    denb = den2_ref[0] + den2_ref[1]
    agg = (num2_ref[0] + num2_ref[1]) / (denb + 1e-16)
    t = x_ref[...] + jnp.dot(agg, wo_ref[...],
                             preferred_element_type=jnp.float32)
    x1 = _bn_exact(t, g1_ref[...], b1_ref[...])
    h = jnp.maximum(
        jnp.dot(x1, w1_ref[...], preferred_element_type=jnp.float32)
        + bb1_ref[...], 0.0)
    t = x1 + jnp.dot(h, w2_ref[...],
                     preferred_element_type=jnp.float32) + bb2_ref[...]
    x2 = _bn_exact(t, g2_ref[...], b2_ref[...])
    x_out_ref[...] = x2
    q_ref[...] = jnp.dot(x2, wq_ref[...], preferred_element_type=jnp.float32)
    k_ref[...] = jnp.dot(x2, wk_ref[...], preferred_element_type=jnp.float32)
    v_ref[...] = jnp.dot(x2, wv_ref[...], preferred_element_type=jnp.float32)


def node_update(x, num, den, wo, g1, b1, w1, bb1, w2, bb2, g2, b2, wq, wk, wv):
    return pl.pallas_call(
        _node_update_body,
        out_shape=[jax.ShapeDtypeStruct((N, D), jnp.float32)] * 4,
    )(x, num, den, wo, g1, b1, w1, bb1, w2, bb2, g2, b2, wq, wk, wv)


def _node_final_body(x_ref, num2_ref, den2_ref, batch_ref, wo_ref, g1_ref,
                     b1_ref, w1_ref, bb1_ref, w2_ref, bb2_ref, g2_ref, b2_ref,
                     mw1_ref, mb1_ref, mw2_ref, mb2_ref,
                     vw1_ref, vb1_ref, vw2_ref, vb2_ref,
                     mu_ref, std_ref):
    denb = den2_ref[0] + den2_ref[1]
    agg = (num2_ref[0] + num2_ref[1]) / (denb + 1e-16)
    t = x_ref[...] + jnp.dot(agg, wo_ref[...],
                             preferred_element_type=jnp.float32)
    x1 = _bn_exact(t, g1_ref[...], b1_ref[...])
    h = jnp.maximum(
        jnp.dot(x1, w1_ref[...], preferred_element_type=jnp.float32)
        + bb1_ref[...], 0.0)
    t = x1 + jnp.dot(h, w2_ref[...],
                     preferred_element_type=jnp.float32) + bb2_ref[...]
    x2 = _bn_exact(t, g2_ref[...], b2_ref[...])
    gi = jax.lax.broadcasted_iota(jnp.int32, (N, G), 1)
    onehot = (batch_ref[...] == gi).astype(jnp.float32)
    pooled = jax.lax.dot_general(
        onehot, x2, (((0,), (0,)), ((), ())),
        preferred_element_type=jnp.float32)
    hm = jnp.maximum(
        jnp.dot(pooled, mw1_ref[...], preferred_element_type=jnp.float32)
        + mb1_ref[...], 0.0)
    mu = jnp.dot(hm, mw2_ref[...],
                 preferred_element_type=jnp.float32) + mb2_ref[...]
    hv = jnp.maximum(
        jnp.dot(pooled, vw1_ref[...], preferred_element_type=jnp.float32)
        + vb1_ref[...], 0.0)
    lv = jnp.dot(hv, vw2_ref[...],
                 preferred_element_type=jnp.float32) + vb2_ref[...]
    mu_ref[...] = mu
    std_ref[...] = jnp.exp(0.5 * lv)


def node_final(x, num, den, batch2d, wo, g1, b1, w1, bb1, w2, bb2, g2, b2,
               mw1, mb1, mw2, mb2, vw1, vb1, vw2, vb2):
    return pl.pallas_call(
        _node_final_body,
        out_shape=[jax.ShapeDtypeStruct((G, 1), jnp.float32)] * 2,
    )(x, num, den, batch2d, wo, g1, b1, w1, bb1, w2, bb2, g2, b2,
      mw1, mb1, mw2, mb2, vw1, vb1, vw2, vb2)


def _stats_to_affine(st, cnt, g, b):
    s, ss = st[0], st[1]
    m = s / cnt
    v = ss / cnt - m * m
    sc = g * jax.lax.rsqrt(v + 1e-5)
    return jnp.stack([sc, b - m * sc])


def kernel(x_cat, x_cont, edge_index, edge_attr, pe, batch, params):
    p = params
    src = edge_index[0]
    dst = edge_index[1]

    t0t, t1t = edge_tables(p['edge_emb'][0], p['edge_emb'][1],
                           p['edge_lin_W'], p['edge_lin_b'][None])
    t2 = sc_embed(t0t, t1t, edge_attr[:, 0], edge_attr[:, 1])

    xe = jnp.concatenate([p['node_emb'][i][x_cat[:, i]] for i in range(3)],
                         axis=-1)
    xf = jnp.concatenate([xe, x_cont], axis=-1)
    x, q, k, v = node_prologue(
        xf, pe, p['node_lin_W'], p['node_lin_b'][None], p['pe_W'],
        p['Wq'][0], p['Wk'][0], p['Wv'][0])

    aff = jnp.concatenate([jnp.ones((1, D), jnp.float32),
                           jnp.zeros((1, D), jnp.float32)])

    for l in range(L):
        eprm = epr_kernel(t2, aff, p['We'][l])
        qkpre = sc_qkv_gather(q, k, src, dst)
        exb, t1, st1 = score_fuse(qkpre, eprm, t2, aff, p['Woe'][l])
        num2, den2 = sc_scatter_acc(exb, dst, src, v)
        aff1 = _stats_to_affine(st1, float(E), p['ebn1_g'][l], p['ebn1_b'][l])
        if l < L - 1:
            t2, st2 = edge_ffn(t1, aff1, p['We1'][l], p['eb1'][l][None],
                               p['We2'][l], p['eb2'][l][None])
            aff = _stats_to_affine(st2, float(E), p['ebn2_g'][l],
                                   p['ebn2_b'][l])
        if l < L - 1:
            x, q, k, v = node_update(
                x, num2, den2, p['Wo'][l], p['bn1_g'][l][None],
                p['bn1_b'][l][None], p['W1'][l], p['b1'][l][None],
                p['W2'][l], p['b2'][l][None], p['bn2_g'][l][None],
                p['bn2_b'][l][None], p['Wq'][l + 1], p['Wk'][l + 1],
                p['Wv'][l + 1])
        else:
            mu, std = node_final(
                x, num2, den2, batch[:, None], p['Wo'][l], p['bn1_g'][l][None],
                p['bn1_b'][l][None], p['W1'][l], p['b1'][l][None],
                p['W2'][l], p['b2'][l][None], p['bn2_g'][l][None],
                p['bn2_b'][l][None],
                p['mW1'], p['mb1'][None], p['mW2'], p['mb2'][None],
                p['vW1'], p['vb1'][None], p['vW2'], p['vb2'][None])
    return (mu, std)


# speedup vs baseline: 30.8829x; 1.0164x over previous
import functools
import jax
import jax.numpy as jnp
from jax import lax
from jax.experimental import pallas as pl
from jax.experimental.pallas import tpu as pltpu, tpu_sc as plsc

N, E, G = 10000, 320000, 128
H, DH, D, L = 8, 16, 128, 4
BE = 2000
SCALE = 0.25

NC, NS = 2, 16
NW = NC * NS
CB = 128
NCH = E // CB
SC_ITERS = -(-NCH // NW)
NPT = N // NS
NPT_A = 624
CBN = 80
NCHN = N // CBN
SCN_ITERS = -(-NCHN // NW)

_sc_mesh = plsc.VectorSubcoreMesh(core_axis_name="c", subcore_axis_name="s")


def _wid():
    return lax.axis_index("s") * NC + lax.axis_index("c")


def _group_mat(dtype=jnp.float32):
    d_i = jax.lax.broadcasted_iota(jnp.int32, (D, H), 0)
    h_i = jax.lax.broadcasted_iota(jnp.int32, (D, H), 1)
    return (d_i // DH == h_i).astype(dtype)


def _sc_qkv_gather_body(q_hbm, k_hbm, src_hbm, dst_hbm,
                        qk_hbm,
                        srcbA, dstbA, qbA, kbA, semA,
                        srcbB, dstbB, qbB, kbB, semB):
    w = _wid()

    def issue(t, srcb, dstb, qb, kb, sem):
        sl = pl.ds(t * CB, CB)
        pltpu.sync_copy(src_hbm.at[sl], srcb)
        pltpu.sync_copy(dst_hbm.at[sl], dstb)
        pltpu.async_copy(q_hbm.at[dstb], qb, sem)
        pltpu.async_copy(k_hbm.at[srcb], kb, sem)

    def finish(t, srcb, dstb, qb, kb, sem):
        pltpu.make_async_copy(q_hbm.at[dstb], qb, sem).wait()
        pltpu.make_async_copy(k_hbm.at[srcb], kb, sem).wait()

        def mul_row(r, _):
            for j in range(D // 16):
                cs = pl.ds(j * 16, 16)
                qb[r, cs] = qb[r, cs] * kb[r, cs]
            return _

        lax.fori_loop(0, CB, mul_row, None)
        pltpu.sync_copy(qb, qk_hbm.at[pl.ds(t * CB, CB)])

    @pl.when(w < NCH)
    def _():
        issue(w, srcbA, dstbA, qbA, kbA, semA)

    def step(i, _):
        t = w + NW * i
        tn = t + NW

        @pl.when(i % 2 == 0)
        def _():
            @pl.when(tn < NCH)
            def _():
                issue(tn, srcbB, dstbB, qbB, kbB, semB)

            @pl.when(t < NCH)
            def _():
                finish(t, srcbA, dstbA, qbA, kbA, semA)

        @pl.when(i % 2 == 1)
        def _():
            @pl.when(tn < NCH)
            def _():
                issue(tn, srcbA, dstbA, qbA, kbA, semA)

            @pl.when(t < NCH)
            def _():
                finish(t, srcbB, dstbB, qbB, kbB, semB)

        return _

    lax.fori_loop(0, SC_ITERS, step, None)


@functools.partial(
    pl.kernel,
    out_type=jax.ShapeDtypeStruct((E, D), jnp.float32),
    mesh=_sc_mesh,
    scratch_types=[
        pltpu.VMEM((CB,), jnp.int32),
        pltpu.VMEM((CB,), jnp.int32),
        pltpu.VMEM((CB, D), jnp.float32),
        pltpu.VMEM((CB, D), jnp.float32),
        pltpu.SemaphoreType.DMA,
        pltpu.VMEM((CB,), jnp.int32),
        pltpu.VMEM((CB,), jnp.int32),
        pltpu.VMEM((CB, D), jnp.float32),
        pltpu.VMEM((CB, D), jnp.float32),
        pltpu.SemaphoreType.DMA,
    ],
)
def sc_qkv_gather(*refs):
    _sc_qkv_gather_body(*refs)


_NZC = N // CB
_NZT = N - _NZC * CB


def _spmem_zero(s, acc_sh, zbuf):
    def zstep(i, _):
        m = s + NS * i

        @pl.when(m < _NZC)
        def _():
            pltpu.sync_copy(zbuf, acc_sh.at[pl.ds(m * CB, CB)])

        @pl.when(m == _NZC)
        def _():
            pltpu.sync_copy(zbuf.at[pl.ds(0, _NZT)],
                            acc_sh.at[pl.ds(_NZC * CB, _NZT)])

        return _

    lax.fori_loop(0, -(-(_NZC + 1) // NS), zstep, None)


def _spmem_dump(c, s, acc_sh, out_hbm, vbuf):
    def dstep(i, _):
        m = s + NS * i

        @pl.when(m < _NZC)
        def _():
            sl = pl.ds(m * CB, CB)
            pltpu.sync_copy(acc_sh.at[sl], vbuf)
            pltpu.sync_copy(vbuf, out_hbm.at[c, sl])

        @pl.when(m == _NZC)
        def _():
            tl = pl.ds(_NZC * CB, _NZT)
            pltpu.sync_copy(acc_sh.at[tl], vbuf.at[pl.ds(0, _NZT)])
            pltpu.sync_copy(vbuf.at[pl.ds(0, _NZT)], out_hbm.at[c, tl])

        return _

    lax.fori_loop(0, -(-(_NZC + 1) // NS), dstep, None)


def _sc_scatter_body(exb_hbm, dst_hbm, src_hbm, v_hbm,
                     num2_hbm, den2_hbm,
                     dstb, srcb, exbb, vb, zbuf, acc_sh, sem):
    c = lax.axis_index("c")
    s = lax.axis_index("s")

    def fill_zero(r, _):
        for j in range(D // 16):
            zbuf[r, pl.ds(j * 16, 16)] = jnp.zeros((16,), jnp.float32)
        return _

    lax.fori_loop(0, CB, fill_zero, None)
    nch_core = NCH // NC
    n_it = -(-nch_core // NS)

    _spmem_zero(s, acc_sh, zbuf)
    plsc.subcore_barrier()

    def step1(i, _):
        tt = s + NS * i

        @pl.when(tt < nch_core)
        def _():
            sl = pl.ds((c * nch_core + tt) * CB, CB)
            pltpu.sync_copy(src_hbm.at[sl], srcb)
            dg = pltpu.async_copy(v_hbm.at[srcb], vb, sem)
            pltpu.sync_copy(dst_hbm.at[sl], dstb)
            pltpu.sync_copy(exb_hbm.at[sl], exbb)
            dg.wait()

            def mul_row(r, _):
                for j in range(D // 16):
                    cs = pl.ds(j * 16, 16)
                    vb[r, cs] = vb[r, cs] * exbb[r, cs]
                return _

            lax.fori_loop(0, CB, mul_row, None)
            pltpu.sync_copy(vb, acc_sh.at[dstb], add=True)

        return _

    lax.fori_loop(0, n_it, step1, None)
    plsc.subcore_barrier()
    _spmem_dump(c, s, acc_sh, num2_hbm, vb)
    _spmem_zero(s, acc_sh, zbuf)
    plsc.subcore_barrier()

    def step2(i, _):
        tt = s + NS * i

        @pl.when(tt < nch_core)
        def _():
            sl = pl.ds((c * nch_core + tt) * CB, CB)
            pltpu.sync_copy(dst_hbm.at[sl], dstb)
            pltpu.sync_copy(exb_hbm.at[sl], exbb)
            pltpu.sync_copy(exbb, acc_sh.at[dstb], add=True)

        return _

    lax.fori_loop(0, n_it, step2, None)
    plsc.subcore_barrier()
    _spmem_dump(c, s, acc_sh, den2_hbm, vb)


@functools.partial(
    pl.kernel,
    out_type=[jax.ShapeDtypeStruct((NC, N, D), jnp.float32),
              jax.ShapeDtypeStruct((NC, N, D), jnp.float32)],
    mesh=_sc_mesh,
    scratch_types=[
        pltpu.VMEM((CB,), jnp.int32),
        pltpu.VMEM((CB,), jnp.int32),
        pltpu.VMEM((CB, D), jnp.float32),
        pltpu.VMEM((CB, D), jnp.float32),
        pltpu.VMEM((CB, D), jnp.float32),
        pltpu.VMEM_SHARED((N, D), jnp.float32),
        pltpu.SemaphoreType.DMA,
    ],
)
def sc_scatter_acc(*refs):
    _sc_scatter_body(*refs)


def _sc_embed_body(t0_hbm, t1_hbm, ea0_hbm, ea1_hbm,
                   e0_hbm,
                   eib0, eib1, b0, b1, sem):
    w = _wid()

    def estep(i, _):
        t = w + NW * i

        @pl.when(t < NCH)
        def _():
            sl = pl.ds(t * CB, CB)
            pltpu.sync_copy(ea0_hbm.at[sl], eib0)
            pltpu.sync_copy(ea1_hbm.at[sl], eib1)
            d0 = pltpu.async_copy(t0_hbm.at[eib0], b0, sem)
            d1 = pltpu.async_copy(t1_hbm.at[eib1], b1, sem)
            d0.wait()
            d1.wait()

            def add_row(r, _):
                for j in range(D // 16):
                    cs = pl.ds(j * 16, 16)
                    b0[r, cs] = b0[r, cs] + b1[r, cs]
                return _

            lax.fori_loop(0, CB, add_row, None)
            pltpu.sync_copy(b0, e0_hbm.at[sl])

        return _

    lax.fori_loop(0, SC_ITERS, estep, None)


@functools.partial(
    pl.kernel,
    out_type=jax.ShapeDtypeStruct((E, D), jnp.float32),
    mesh=_sc_mesh,
    scratch_types=[
        pltpu.VMEM((CB,), jnp.int32),
        pltpu.VMEM((CB,), jnp.int32),
        pltpu.VMEM((CB, D), jnp.float32),
        pltpu.VMEM((CB, D), jnp.float32),
        pltpu.SemaphoreType.DMA,
    ],
)
def sc_embed(*refs):
    _sc_embed_body(*refs)


def _edge_tables_body(ee0_ref, ee1_ref, w_ref, b_ref, t0_ref, t1_ref):
    t0_ref[...] = jnp.dot(ee0_ref[...], w_ref[0:8, :],
                          preferred_element_type=jnp.float32) + 0.5 * b_ref[...]
    t1_ref[...] = jnp.dot(ee1_ref[...], w_ref[8:16, :],
                          preferred_element_type=jnp.float32) + 0.5 * b_ref[...]


def edge_tables(ee0, ee1, w, b):
    return pl.pallas_call(
        _edge_tables_body,
        out_shape=[jax.ShapeDtypeStruct((1000, D), jnp.float32)] * 2,
    )(ee0, ee1, w, b)


def _epr_body(t2_ref, aff_ref, w_ref, out_ref):
    el = t2_ref[...] * aff_ref[0:1, :] + aff_ref[1:2, :]
    out_ref[...] = jnp.dot(el, w_ref[...], preferred_element_type=jnp.float32)


def epr_kernel(t2, aff, w):
    return pl.pallas_call(
        _epr_body,
        grid=(E // BE,),
        in_specs=[
            pl.BlockSpec((BE, D), lambda i: (i, 0)),
            pl.BlockSpec((2, D), lambda i: (0, 0)),
            pl.BlockSpec((D, D), lambda i: (0, 0)),
        ],
        out_specs=pl.BlockSpec((BE, D), lambda i: (i, 0)),
        out_shape=jax.ShapeDtypeStruct((E, D), jnp.float32),
    )(t2, aff, w)


def _score_fuse_body(qkpre_ref, epr_ref, t2_ref, aff_ref, woe_ref,
                     exb_ref, t1_ref, st_ref):
    qk = qkpre_ref[...] * epr_ref[...] * SCALE
    gm = _group_mat()
    score = jnp.dot(qk, gm, preferred_element_type=jnp.float32)
    ex = jnp.exp(score)
    exb = jnp.dot(ex, gm.T, preferred_element_type=jnp.float32)
    exb_ref[...] = exb
    el = t2_ref[...] * aff_ref[0:1, :] + aff_ref[1:2, :]
    t1 = el + jnp.dot(qk, woe_ref[...], preferred_element_type=jnp.float32)
    t1_ref[...] = t1
    s = jnp.sum(t1, axis=0, keepdims=True)
    ss = jnp.sum(t1 * t1, axis=0, keepdims=True)
    blk = jnp.concatenate(
        [s, ss, jnp.zeros((6, D), jnp.float32)], axis=0)

    @pl.when(pl.program_id(0) == 0)
    def _():
        st_ref[...] = jnp.zeros_like(st_ref)

    st_ref[...] += blk


def score_fuse(qkpre, epr, t2, aff, woe):
    return pl.pallas_call(
        _score_fuse_body,
        grid=(E // BE,),
        in_specs=[
            pl.BlockSpec((BE, D), lambda i: (i, 0)),
            pl.BlockSpec((BE, D), lambda i: (i, 0)),
            pl.BlockSpec((BE, D), lambda i: (i, 0)),
            pl.BlockSpec((2, D), lambda i: (0, 0)),
            pl.BlockSpec((D, D), lambda i: (0, 0)),
        ],
        out_specs=[
            pl.BlockSpec((BE, D), lambda i: (i, 0)),
            pl.BlockSpec((BE, D), lambda i: (i, 0)),
            pl.BlockSpec((8, D), lambda i: (0, 0)),
        ],
        out_shape=[
            jax.ShapeDtypeStruct((E, D), jnp.float32),
            jax.ShapeDtypeStruct((E, D), jnp.float32),
            jax.ShapeDtypeStruct((8, D), jnp.float32),
        ],
    )(qkpre, epr, t2, aff, woe)


def _edge_ffn_body(t1_ref, aff_ref, w1_ref, b1_ref, w2_ref, b2_ref,
                   t2_ref, st_ref):
    ep = t1_ref[...] * aff_ref[0:1, :] + aff_ref[1:2, :]
    he = jnp.maximum(
        jnp.dot(ep, w1_ref[...], preferred_element_type=jnp.float32)
        + b1_ref[...], 0.0)
    t2 = ep + jnp.dot(he, w2_ref[...],
                      preferred_element_type=jnp.float32) + b2_ref[...]
    t2_ref[...] = t2
    s = jnp.sum(t2, axis=0, keepdims=True)
    ss = jnp.sum(t2 * t2, axis=0, keepdims=True)
    blk = jnp.concatenate([s, ss, jnp.zeros((6, D), jnp.float32)], axis=0)

    @pl.when(pl.program_id(0) == 0)
    def _():
        st_ref[...] = jnp.zeros_like(st_ref)

    st_ref[...] += blk


def edge_ffn(t1, aff, w1, b1, w2, b2):
    return pl.pallas_call(
        _edge_ffn_body,
        grid=(E // BE,),
        in_specs=[
            pl.BlockSpec((BE, D), lambda i: (i, 0)),
            pl.BlockSpec((2, D), lambda i: (0, 0)),
            pl.BlockSpec((D, 2 * D), lambda i: (0, 0)),
            pl.BlockSpec((1, 2 * D), lambda i: (0, 0)),
            pl.BlockSpec((2 * D, D), lambda i: (0, 0)),
            pl.BlockSpec((1, D), lambda i: (0, 0)),
        ],
        out_specs=[
            pl.BlockSpec((BE, D), lambda i: (i, 0)),
            pl.BlockSpec((8, D), lambda i: (0, 0)),
        ],
        out_shape=[
            jax.ShapeDtypeStruct((E, D), jnp.float32),
            jax.ShapeDtypeStruct((8, D), jnp.float32),
        ],
    )(t1, aff, w1, b1, w2, b2)


def _bn_exact(t, g, b):
    m = jnp.mean(t, axis=0, keepdims=True)
    v = jnp.mean(t * t, axis=0, keepdims=True) - m * m
    return (t - m) * jax.lax.rsqrt(v + 1e-5) * g + b


def _node_prologue_body(xf_ref, pe_ref, wn_ref, bn_ref, wpe_ref,
                        wq_ref, wk_ref, wv_ref,
                        x_ref, q_ref, k_ref, v_ref):
    x = (jnp.dot(xf_ref[...], wn_ref[...], preferred_element_type=jnp.float32)
         + bn_ref[...]
         + jnp.dot(pe_ref[...], wpe_ref[...],
                   preferred_element_type=jnp.float32))
    x_ref[...] = x
    q_ref[...] = jnp.dot(x, wq_ref[...], preferred_element_type=jnp.float32)
    k_ref[...] = jnp.dot(x, wk_ref[...], preferred_element_type=jnp.float32)
    v_ref[...] = jnp.dot(x, wv_ref[...], preferred_element_type=jnp.float32)


def node_prologue(xf, pe, wn, bn, wpe, wq, wk, wv):
    return pl.pallas_call(
        _node_prologue_body,
        out_shape=[jax.ShapeDtypeStruct((N, D), jnp.float32)] * 4,
    )(xf, pe, wn, bn, wpe, wq, wk, wv)


def _node_update_body(x_ref, num2_ref, den2_ref, wo_ref, g1_ref, b1_ref,
                      w1_ref, bb1_ref, w2_ref, bb2_ref, g2_ref, b2_ref,
                      wq_ref, wk_ref, wv_ref,
                      x_out_ref, q_ref, k_ref, v_ref):
    denb = den2_ref[0] + den2_ref[1]
    agg = (num2_ref[0] + num2_ref[1]) / (denb + 1e-16)
    t = x_ref[...] + jnp.dot(agg, wo_ref[...],
                             preferred_element_type=jnp.float32)
    x1 = _bn_exact(t, g1_ref[...], b1_ref[...])
    h = jnp.maximum(
        jnp.dot(x1, w1_ref[...], preferred_element_type=jnp.float32)
        + bb1_ref[...], 0.0)
    t = x1 + jnp.dot(h, w2_ref[...],
                     preferred_element_type=jnp.float32) + bb2_ref[...]
    x2 = _bn_exact(t, g2_ref[...], b2_ref[...])
    x_out_ref[...] = x2
    q_ref[...] = jnp.dot(x2, wq_ref[...], preferred_element_type=jnp.float32)
    k_ref[...] = jnp.dot(x2, wk_ref[...], preferred_element_type=jnp.float32)
    v_ref[...] = jnp.dot(x2, wv_ref[...], preferred_element_type=jnp.float32)


def node_update(x, num, den, wo, g1, b1, w1, bb1, w2, bb2, g2, b2, wq, wk, wv):
    return pl.pallas_call(
        _node_update_body,
        out_shape=[jax.ShapeDtypeStruct((N, D), jnp.float32)] * 4,
    )(x, num, den, wo, g1, b1, w1, bb1, w2, bb2, g2, b2, wq, wk, wv)


def _node_final_body(x_ref, num2_ref, den2_ref, batch_ref, wo_ref, g1_ref,
                     b1_ref, w1_ref, bb1_ref, w2_ref, bb2_ref, g2_ref, b2_ref,
                     mw1_ref, mb1_ref, mw2_ref, mb2_ref,
                     vw1_ref, vb1_ref, vw2_ref, vb2_ref,
                     mu_ref, std_ref):
    denb = den2_ref[0] + den2_ref[1]
    agg = (num2_ref[0] + num2_ref[1]) / (denb + 1e-16)
    t = x_ref[...] + jnp.dot(agg, wo_ref[...],
                             preferred_element_type=jnp.float32)
    x1 = _bn_exact(t, g1_ref[...], b1_ref[...])
    h = jnp.maximum(
        jnp.dot(x1, w1_ref[...], preferred_element_type=jnp.float32)
        + bb1_ref[...], 0.0)
    t = x1 + jnp.dot(h, w2_ref[...],
                     preferred_element_type=jnp.float32) + bb2_ref[...]
    x2 = _bn_exact(t, g2_ref[...], b2_ref[...])
    gi = jax.lax.broadcasted_iota(jnp.int32, (N, G), 1)
    onehot = (batch_ref[...] == gi).astype(jnp.float32)
    pooled = jax.lax.dot_general(
        onehot, x2, (((0,), (0,)), ((), ())),
        preferred_element_type=jnp.float32)
    hm = jnp.maximum(
        jnp.dot(pooled, mw1_ref[...], preferred_element_type=jnp.float32)
        + mb1_ref[...], 0.0)
    mu = jnp.dot(hm, mw2_ref[...],
                 preferred_element_type=jnp.float32) + mb2_ref[...]
    hv = jnp.maximum(
        jnp.dot(pooled, vw1_ref[...], preferred_element_type=jnp.float32)
        + vb1_ref[...], 0.0)
    lv = jnp.dot(hv, vw2_ref[...],
                 preferred_element_type=jnp.float32) + vb2_ref[...]
    mu_ref[...] = mu
    std_ref[...] = jnp.exp(0.5 * lv)


def node_final(x, num, den, batch2d, wo, g1, b1, w1, bb1, w2, bb2, g2, b2,
               mw1, mb1, mw2, mb2, vw1, vb1, vw2, vb2):
    return pl.pallas_call(
        _node_final_body,
        out_shape=[jax.ShapeDtypeStruct((G, 1), jnp.float32)] * 2,
    )(x, num, den, batch2d, wo, g1, b1, w1, bb1, w2, bb2, g2, b2,
      mw1, mb1, mw2, mb2, vw1, vb1, vw2, vb2)


def _stats_to_affine(st, cnt, g, b):
    s, ss = st[0], st[1]
    m = s / cnt
    v = ss / cnt - m * m
    sc = g * jax.lax.rsqrt(v + 1e-5)
    return jnp.stack([sc, b - m * sc])


def kernel(x_cat, x_cont, edge_index, edge_attr, pe, batch, params):
    p = params
    src = edge_index[0]
    dst = edge_index[1]

    t0t, t1t = edge_tables(p['edge_emb'][0], p['edge_emb'][1],
                           p['edge_lin_W'], p['edge_lin_b'][None])
    t2 = sc_embed(t0t, t1t, edge_attr[:, 0], edge_attr[:, 1])

    xe = jnp.concatenate([p['node_emb'][i][x_cat[:, i]] for i in range(3)],
                         axis=-1)
    xf = jnp.concatenate([xe, x_cont], axis=-1)
    x, q, k, v = node_prologue(
        xf, pe, p['node_lin_W'], p['node_lin_b'][None], p['pe_W'],
        p['Wq'][0], p['Wk'][0], p['Wv'][0])

    aff = jnp.concatenate([jnp.ones((1, D), jnp.float32),
                           jnp.zeros((1, D), jnp.float32)])

    for l in range(L):
        eprm = epr_kernel(t2, aff, p['We'][l])
        qkpre = sc_qkv_gather(q, k, src, dst)
        exb, t1, st1 = score_fuse(qkpre, eprm, t2, aff, p['Woe'][l])
        num2, den2 = sc_scatter_acc(exb, dst, src, v)
        aff1 = _stats_to_affine(st1, float(E), p['ebn1_g'][l], p['ebn1_b'][l])
        if l < L - 1:
            t2, st2 = edge_ffn(t1, aff1, p['We1'][l], p['eb1'][l][None],
                               p['We2'][l], p['eb2'][l][None])
            aff = _stats_to_affine(st2, float(E), p['ebn2_g'][l],
                                   p['ebn2_b'][l])
        if l < L - 1:
            x, q, k, v = node_update(
                x, num2, den2, p['Wo'][l], p['bn1_g'][l][None],
                p['bn1_b'][l][None], p['W1'][l], p['b1'][l][None],
                p['W2'][l], p['b2'][l][None], p['bn2_g'][l][None],
                p['bn2_b'][l][None], p['Wq'][l + 1], p['Wk'][l + 1],
                p['Wv'][l + 1])
        else:
            mu, std = node_final(
                x, num2, den2, batch[:, None], p['Wo'][l], p['bn1_g'][l][None],
                p['bn1_b'][l][None], p['W1'][l], p['b1'][l][None],
                p['W2'][l], p['b2'][l][None], p['bn2_g'][l][None],
                p['bn2_b'][l][None],
                p['mW1'], p['mb1'][None], p['mW2'], p['mb2'][None],
                p['vW1'], p['vb1'][None], p['vW2'], p['vb2'][None])
    return (mu, std)
